# baseline (speedup 1.0000x reference)
import functools

import jax
import jax.numpy as jnp
from jax import lax
from jax.experimental import pallas as pl
from jax.experimental.pallas import tpu as pltpu
from jax.experimental.pallas import tpu_sc as plsc

N = 10000
E = 160000
D = 256
H = 128
EPS = 1e-5

C = 64
TILES = 16
EDGES_PER_TILE = 10112
EP = EDGES_PER_TILE * TILES
P = EP - E
CHUNKS = EDGES_PER_TILE // C
NACC = N + 16


def _mm_kernel(x_ref, w_ref, o_ref):
    o_ref[...] = jnp.dot(x_ref[...], w_ref[...],
                         preferred_element_type=jnp.float32)


def _node_matmuls(n_feat, wnt):
    return pl.pallas_call(
        _mm_kernel,
        grid=(25,),
        in_specs=[pl.BlockSpec((400, D), lambda i: (i, 0)),
                  pl.BlockSpec((D, 4 * D), lambda i: (0, 0))],
        out_specs=pl.BlockSpec((400, 4 * D), lambda i: (i, 0)),
        out_shape=jax.ShapeDtypeStruct((N, 4 * D), jnp.float32),
    )(n_feat, wnt)


def _edge_mm(e_feat, w4t):
    return pl.pallas_call(
        _mm_kernel,
        grid=(160,),
        in_specs=[pl.BlockSpec((1000, D), lambda i: (i, 0)),
                  pl.BlockSpec((D, D), lambda i: (0, 0))],
        out_specs=pl.BlockSpec((1000, D), lambda i: (i, 0)),
        out_shape=jax.ShapeDtypeStruct((E, D), jnp.float32),
    )(e_feat, w4t)


def _exp_kernel(x_ref, ex_ref):
    ex_ref[...] = jnp.exp(x_ref[...])


def _edge_exp(e_feat):
    return pl.pallas_call(
        _exp_kernel,
        grid=(160,),
        in_specs=[pl.BlockSpec((1000, D), lambda i: (i, 0))],
        out_specs=pl.BlockSpec((1000, D), lambda i: (i, 0)),
        out_shape=jax.ShapeDtypeStruct((EP, D), jnp.float32),
    )(e_feat)


def _estats_kernel(ep_ref, w4_ref, st_ref):
    s = ep_ref[...] + w4_ref[...]
    ps = jnp.sum(s, axis=0)
    pq = jnp.sum(s * s, axis=0)
    z = jnp.zeros((6, D), jnp.float32)
    st_ref[...] = jnp.concatenate([ps[None], pq[None], z], axis=0)


def _edge_stats(e_pre, w4e):
    return pl.pallas_call(
        _estats_kernel,
        grid=(160,),
        in_specs=[pl.BlockSpec((1000, D), lambda i: (i, 0)),
                  pl.BlockSpec((1000, D), lambda i: (i, 0))],
        out_specs=pl.BlockSpec((8, D), lambda i: (i, 0)),
        out_shape=jax.ShapeDtypeStruct((160 * 8, D), jnp.float32),
    )(e_pre, w4e)


def _newe_kernel(st_ref, ep_ref, w4_ref, ef_ref, g_ref, b_ref, o_ref):
    st = jnp.sum(st_ref[...].reshape(160, 8, D), axis=0)
    mean = st[0:1] / E
    var = st[1:2] / E - mean * mean
    inv = lax.rsqrt(var + EPS)
    s = ep_ref[...] + w4_ref[...]
    xn = (s - mean) * inv * g_ref[...] + b_ref[...]
    o_ref[...] = jnp.maximum(xn, 0.0) + ef_ref[...]


def _edge_update(stats, e_pre, w4e, e_feat, gamma_e, beta_e):
    return pl.pallas_call(
        _newe_kernel,
        grid=(160,),
        in_specs=[pl.BlockSpec((160 * 8, D), lambda i: (0, 0)),
                  pl.BlockSpec((1000, D), lambda i: (i, 0)),
                  pl.BlockSpec((1000, D), lambda i: (i, 0)),
                  pl.BlockSpec((1000, D), lambda i: (i, 0)),
                  pl.BlockSpec((1, D), lambda i: (0, 0)),
                  pl.BlockSpec((1, D), lambda i: (0, 0))],
        out_specs=pl.BlockSpec((1000, D), lambda i: (i, 0)),
        out_shape=jax.ShapeDtypeStruct((E, D), jnp.float32),
    )(stats, e_pre, w4e, e_feat, gamma_e, beta_e)


def _nstats_kernel(na_ref, dn_ref, w2_ref, st_ref):
    s = na_ref[...] / jnp.maximum(dn_ref[...], 1e-30) + w2_ref[...]
    ps = jnp.sum(s, axis=0)
    pq = jnp.sum(s * s, axis=0)
    z = jnp.zeros((6, D), jnp.float32)
    st_ref[...] = jnp.concatenate([ps[None], pq[None], z], axis=0)


def _node_stats(nacc, denom, w2h):
    return pl.pallas_call(
        _nstats_kernel,
        grid=(10,),
        in_specs=[pl.BlockSpec((1000, D), lambda i: (i, 0)),
                  pl.BlockSpec((1000, D), lambda i: (i, 0)),
                  pl.BlockSpec((1000, D), lambda i: (i, 0))],
        out_specs=pl.BlockSpec((8, D), lambda i: (i, 0)),
        out_shape=jax.ShapeDtypeStruct((10 * 8, D), jnp.float32),
    )(nacc, denom, w2h)


def _newh_kernel(st_ref, na_ref, dn_ref, w2_ref, nf_ref, g_ref, b_ref, o_ref):
    st = jnp.sum(st_ref[...].reshape(10, 8, D), axis=0)
    mean = st[0:1] / N
    var = st[1:2] / N - mean * mean
    inv = lax.rsqrt(var + EPS)
    s = na_ref[...] / jnp.maximum(dn_ref[...], 1e-30) + w2_ref[...]
    xn = (s - mean) * inv * g_ref[...] + b_ref[...]
    o_ref[...] = jnp.maximum(xn, 0.0) + nf_ref[...]


def _node_update(nacc, denom, w2h, n_feat, gamma_n, beta_n):
    stats = _node_stats(nacc, denom, w2h)
    return pl.pallas_call(
        _newh_kernel,
        grid=(10,),
        in_specs=[pl.BlockSpec((10 * 8, D), lambda i: (0, 0)),
                  pl.BlockSpec((1000, D), lambda i: (i, 0)),
                  pl.BlockSpec((1000, D), lambda i: (i, 0)),
                  pl.BlockSpec((1000, D), lambda i: (i, 0)),
                  pl.BlockSpec((1000, D), lambda i: (i, 0)),
                  pl.BlockSpec((1, D), lambda i: (0, 0)),
                  pl.BlockSpec((1, D), lambda i: (0, 0))],
        out_specs=pl.BlockSpec((1000, D), lambda i: (i, 0)),
        out_shape=jax.ShapeDtypeStruct((N, D), jnp.float32),
    )(stats, nacc, denom, w2h, n_feat, gamma_n, beta_n)


_MESH = plsc.VectorSubcoreMesh(core_axis_name="c", subcore_axis_name="s")

PAIRS = CHUNKS // 2


def _copy_idx(dst_ref, src_ref):
    for v in range(C // 16):
        sl = pl.ds(v * 16, 16)
        dst_ref[sl] = src_ref[sl]


class _EdgePipe:
    def __init__(self, c, s, sadj, dadj, dstr, ta_hbm, tb_hbm, ex_hbm,
                 sa, da, dv, dv2, b0, b1, be, isem, dsem, osem, acc):
        self.c, self.s = c, s
        self.sadj, self.dadj, self.dstr = sadj, dadj, dstr
        self.ta, self.tb, self.ex = ta_hbm, tb_hbm, ex_hbm
        self.sa, self.da, self.dv, self.dv2 = sa, da, dv, dv2
        self.b0, self.b1, self.be = b0, b1, be
        self.isem, self.dsem, self.osem = isem, dsem, osem
        self.acc = acc

    def _e0(self, k):
        return self.s * EDGES_PER_TILE + k * C

    def _i_descs(self, k, b):
        e0 = self._e0(k)
        ge = self.c * EP + e0
        return [
            (self.sadj.at[pl.ds(ge, C)], self.sa.at[b], self.isem.at[b, 0]),
            (self.dadj.at[pl.ds(ge, C)], self.da.at[b], self.isem.at[b, 1]),
            (self.dstr.at[pl.ds(e0, C)], self.dv.at[b], self.isem.at[b, 2]),
        ]

    def _d_descs(self, k, b):
        e0 = self._e0(k)
        half = self.c * H
        return [
            (self.ta.at[self.sa.at[b]], self.b0.at[b], self.dsem.at[b, 0]),
            (self.tb.at[self.da.at[b]], self.b1.at[b], self.dsem.at[b, 1]),
            (self.ex.at[pl.ds(e0, C), pl.ds(half, H)], self.be.at[b],
             self.dsem.at[b, 2]),
        ]

    def i_start(self, k, b):
        for sd in self._i_descs(k, b):
            pltpu.async_copy(*sd)

    def i_wait(self, k, b):
        for sd in self._i_descs(k, b):
            pltpu.make_async_copy(*sd).wait()

    def d_start(self, k, b):
        for sd in self._d_descs(k, b):
            pltpu.async_copy(*sd)

    def d_wait(self, k, b):
        for sd in self._d_descs(k, b):
            pltpu.make_async_copy(*sd).wait()

    def run(self):
        self.i_start(0, 0)

        def pair(g, _):
            k0 = 2 * g
            self.i_wait(k0, 0)

            @pl.when(g >= 1)
            def _():
                self.o_wait(k0 - 2, 0)
            self.d_start(k0, 0)

            @pl.when(g >= 1)
            def _():
                self.d_wait(k0 - 1, 1)
                self.comp(1)
                self.o_go(k0 - 1, 1)
            self.i_start(k0 + 1, 1)

            k1 = k0 + 1
            self.i_wait(k1, 1)

            @pl.when(g >= 1)
            def _():
                self.o_wait(k1 - 2, 1)
            self.d_start(k1, 1)

            self.d_wait(k0, 0)
            self.comp(0)
            self.o_go(k0, 0)

            @pl.when(k1 + 1 < CHUNKS)
            def _():
                self.i_start(k1 + 1, 0)
            return 0

        lax.fori_loop(0, PAIRS, pair, 0)
        self.d_wait(CHUNKS - 1, 1)
        self.comp(1)
        self.o_go(CHUNKS - 1, 1)
        self.o_wait(CHUNKS - 2, 0)
        self.o_wait(CHUNKS - 1, 1)


class _S1Pipe(_EdgePipe):
    def comp(self, b):
        _copy_idx(self.dv2.at[b], self.dv.at[b])

        @plsc.parallel_loop(0, C, unroll=2)
        def _(r):
            for j in range(H // 16):
                sl = pl.ds(j * 16, 16)
                self.b0[b, r, sl] = self.b0[b, r, sl] + self.b1[b, r, sl]

    def o_go(self, k, b):
        e0 = self._e0(k)
        half = self.c * H
        pltpu.async_copy(self.b0.at[b],
                         self.epre.at[pl.ds(e0, C), pl.ds(half, H)],
                         self.osem.at[b, 0])
        pltpu.async_copy(self.be.at[b], self.acc.at[self.dv2.at[b]],
                         self.osem.at[b, 1], add=True)

    def o_wait(self, k, b):
        e0 = self._e0(k)
        half = self.c * H
        pltpu.make_async_copy(self.b0.at[b],
                              self.epre.at[pl.ds(e0, C), pl.ds(half, H)],
                              self.osem.at[b, 0]).wait()
        pltpu.make_async_copy(self.be.at[b], self.acc.at[self.dv2.at[b]],
                              self.osem.at[b, 1]).wait()


class _S2Pipe(_EdgePipe):
    def _i_descs(self, k, b):
        e0 = self._e0(k)
        ge = self.c * EP + e0
        return [
            (self.sadj.at[pl.ds(ge, C)], self.sa.at[b], self.isem.at[b, 0]),
            (self.dstr.at[pl.ds(e0, C)], self.dv.at[b], self.isem.at[b, 1]),
        ]

    def _d_descs(self, k, b):
        e0 = self._e0(k)
        half = self.c * H
        return [
            (self.ta.at[self.sa.at[b]], self.b0.at[b], self.dsem.at[b, 0]),
            (self.ex.at[pl.ds(e0, C), pl.ds(half, H)], self.be.at[b],
             self.dsem.at[b, 1]),
        ]

    def comp(self, b):
        _copy_idx(self.dv2.at[b], self.dv.at[b])

        @plsc.parallel_loop(0, C, unroll=2)
        def _(r):
            for j in range(H // 16):
                sl = pl.ds(j * 16, 16)
                self.b0[b, r, sl] = self.b0[b, r, sl] * self.be[b, r, sl]

    def o_go(self, k, b):
        pltpu.async_copy(self.b0.at[b], self.acc.at[self.dv2.at[b]],
                         self.osem.at[b, 0], add=True)

    def o_wait(self, k, b):
        pltpu.make_async_copy(self.b0.at[b], self.acc.at[self.dv2.at[b]],
                              self.osem.at[b, 0]).wait()


_SC_SCRATCH = [
    pltpu.VMEM_SHARED((NACC, H), jnp.float32),
    pltpu.VMEM((2, C), jnp.int32),
    pltpu.VMEM((2, C), jnp.int32),
    pltpu.VMEM((2, C), jnp.int32),
    pltpu.VMEM((2, C), jnp.int32),
    pltpu.VMEM((2, C, H), jnp.float32),
    pltpu.VMEM((2, C, H), jnp.float32),
    pltpu.VMEM((2, C, H), jnp.float32),
    pltpu.SemaphoreType.DMA((2, 3)),
    pltpu.SemaphoreType.DMA((2, 3)),
    pltpu.SemaphoreType.DMA((2, 2)),
]


def _sc_edge_kernel(sadj_hbm, dadj_hbm, dst_hbm, w0_hbm, w1_hbm, ex_hbm,
                    zero_hbm, epre_hbm, denom_hbm,
                    acc, sa, da, dv, dv2, b0, b1, be, isem, dsem, osem):
    c = lax.axis_index("c")
    s = lax.axis_index("s")

    @pl.when(s == 0)
    def _():
        pltpu.sync_copy(zero_hbm, acc)
    plsc.subcore_barrier()

    p = _S1Pipe(c, s, sadj_hbm, dadj_hbm, dst_hbm, w0_hbm, w1_hbm, ex_hbm,
                sa, da, dv, dv2, b0, b1, be, isem, dsem, osem, acc)
    p.epre = epre_hbm
    p.run()

    plsc.subcore_barrier()

    @pl.when(s == 0)
    def _():
        pltpu.sync_copy(acc, denom_hbm.at[c])


def _sc_edge(sadj, dadj, dst, w0cat, w1cat, ex, zeros):
    return pl.kernel(
        _sc_edge_kernel,
        out_type=[jax.ShapeDtypeStruct((EP, D), jnp.float32),
                  jax.ShapeDtypeStruct((2, NACC, H), jnp.float32)],
        mesh=_MESH,
        scratch_types=_SC_SCRATCH,
    )(sadj, dadj, dst, w0cat, w1cat, ex, zeros)


def _sc_node_kernel(sadj_hbm, dst_hbm, w3_hbm, ex_hbm, zero_hbm, ntmp_hbm,
                    acc, sa, dv, dv2, b0, be, isem, dsem, osem):
    c = lax.axis_index("c")
    s = lax.axis_index("s")

    @pl.when(s == 0)
    def _():
        pltpu.sync_copy(zero_hbm, acc)
    plsc.subcore_barrier()

    p = _S2Pipe(c, s, sadj_hbm, sadj_hbm, dst_hbm, w3_hbm, w3_hbm, ex_hbm,
                sa, sa, dv, dv2, b0, b0, be, isem, dsem, osem, acc)
    p.run()

    plsc.subcore_barrier()

    @pl.when(s == 0)
    def _():
        pltpu.sync_copy(acc, ntmp_hbm.at[c])


def _sc_node(sadj, dst, w3cat, ex, zeros):
    return pl.kernel(
        _sc_node_kernel,
        out_type=jax.ShapeDtypeStruct((2, NACC, H), jnp.float32),
        mesh=_MESH,
        scratch_types=[
            pltpu.VMEM_SHARED((NACC, H), jnp.float32),
            pltpu.VMEM((2, C), jnp.int32),
            pltpu.VMEM((2, C), jnp.int32),
            pltpu.VMEM((2, C), jnp.int32),
            pltpu.VMEM((2, C, H), jnp.float32),
            pltpu.VMEM((2, C, H), jnp.float32),
            pltpu.SemaphoreType.DMA((2, 2)),
            pltpu.SemaphoreType.DMA((2, 2)),
            pltpu.SemaphoreType.DMA((2, 1)),
        ],
    )(sadj, dst, w3cat, ex, zeros)


def _halves_cat(x):
    return jnp.concatenate([x[:, :H], x[:, H:]], axis=0)


def kernel(n_feat, e_feat, edge_index, W0, W1, W2, W3, W4,
           gamma_e, beta_e, gamma_n, beta_n):
    src = edge_index[0]
    dst = edge_index[1]
    zp = jnp.zeros((P,), jnp.int32)
    sadj = jnp.concatenate([src, zp, src + N, zp + N])
    dadj = jnp.concatenate([dst, zp, dst + N, zp + N])
    dstp = jnp.concatenate([dst, zp + N])

    wnt = jnp.concatenate([W0, W1, W2, W3], axis=0).T
    hcat = _node_matmuls(n_feat, wnt)
    w0h, w1h, w2h, w3h = (hcat[:, :D], hcat[:, D:2 * D],
                          hcat[:, 2 * D:3 * D], hcat[:, 3 * D:])

    ex = _edge_exp(e_feat)

    zeros = jnp.zeros((NACC, H), jnp.float32)
    e_pre, denom = _sc_edge(sadj, dadj, dstp, _halves_cat(w0h),
                            _halves_cat(w1h), ex, zeros)

    w4e = _edge_mm(e_feat, W4.T)

    ntmp_h = _sc_node(sadj, dstp, _halves_cat(w3h), ex, zeros)

    stats = _edge_stats(e_pre, w4e)
    new_e = _edge_update(stats, e_pre, w4e, e_feat,
                         gamma_e.reshape(1, D), beta_e.reshape(1, D))

    nacc = ntmp_h[:, :N, :].transpose(1, 0, 2).reshape(N, D)
    dn = denom[:, :N, :].transpose(1, 0, 2).reshape(N, D)
    new_h = _node_update(nacc, dn, w2h, n_feat,
                         gamma_n.reshape(1, D), beta_n.reshape(1, D))
    return (new_h, new_e)


# speedup vs baseline: 1.1847x; 1.1847x over previous
import functools

import jax
import jax.numpy as jnp
from jax import lax
from jax.experimental import pallas as pl
from jax.experimental.pallas import tpu as pltpu
from jax.experimental.pallas import tpu_sc as plsc

N = 10000
E = 160000
D = 256
H = 128
EPS = 1e-5

C = 64
TILES = 16
EDGES_PER_TILE = 10112
EP = EDGES_PER_TILE * TILES
P = EP - E
CHUNKS = EDGES_PER_TILE // C
NACC = N + 16


def _mm_kernel(x_ref, w_ref, o_ref):
    o_ref[...] = jnp.dot(x_ref[...], w_ref[...],
                         preferred_element_type=jnp.float32)


def _node_matmuls(n_feat, wnt):
    return pl.pallas_call(
        _mm_kernel,
        grid=(25,),
        in_specs=[pl.BlockSpec((400, D), lambda i: (i, 0)),
                  pl.BlockSpec((D, 4 * D), lambda i: (0, 0))],
        out_specs=pl.BlockSpec((400, 4 * D), lambda i: (i, 0)),
        out_shape=jax.ShapeDtypeStruct((N, 4 * D), jnp.float32),
    )(n_feat, wnt)


def _bf16_mm_kernel(x_ref, w_ref, o_ref):
    o_ref[...] = jnp.dot(x_ref[...].astype(jnp.bfloat16), w_ref[...],
                         preferred_element_type=jnp.float32)


def _edge_mm(e_feat, w4t):
    return pl.pallas_call(
        _bf16_mm_kernel,
        grid=(160,),
        in_specs=[pl.BlockSpec((1000, D), lambda i: (i, 0)),
                  pl.BlockSpec((D, D), lambda i: (0, 0))],
        out_specs=pl.BlockSpec((1000, D), lambda i: (i, 0)),
        out_shape=jax.ShapeDtypeStruct((E, D), jnp.float32),
    )(e_feat, w4t.astype(jnp.bfloat16))


def _estats_kernel(ep_ref, w4_ref, st_ref):
    s = ep_ref[...] + w4_ref[...]
    ps = jnp.sum(s, axis=0)
    pq = jnp.sum(s * s, axis=0)
    z = jnp.zeros((6, D), jnp.float32)
    st_ref[...] = jnp.concatenate([ps[None], pq[None], z], axis=0)


def _edge_stats(e_pre, w4e):
    return pl.pallas_call(
        _estats_kernel,
        grid=(160,),
        in_specs=[pl.BlockSpec((1000, D), lambda i: (i, 0)),
                  pl.BlockSpec((1000, D), lambda i: (i, 0))],
        out_specs=pl.BlockSpec((8, D), lambda i: (i, 0)),
        out_shape=jax.ShapeDtypeStruct((160 * 8, D), jnp.float32),
    )(e_pre, w4e)


def _newe_kernel(st_ref, ep_ref, w4_ref, ef_ref, g_ref, b_ref, o_ref):
    st = jnp.sum(st_ref[...].reshape(160, 8, D), axis=0)
    mean = st[0:1] / E
    var = st[1:2] / E - mean * mean
    inv = lax.rsqrt(var + EPS)
    s = ep_ref[...] + w4_ref[...]
    xn = (s - mean) * inv * g_ref[...] + b_ref[...]
    o_ref[...] = jnp.maximum(xn, 0.0) + ef_ref[...]


def _edge_update(stats, e_pre, w4e, e_feat, gamma_e, beta_e):
    return pl.pallas_call(
        _newe_kernel,
        grid=(160,),
        in_specs=[pl.BlockSpec((160 * 8, D), lambda i: (0, 0)),
                  pl.BlockSpec((1000, D), lambda i: (i, 0)),
                  pl.BlockSpec((1000, D), lambda i: (i, 0)),
                  pl.BlockSpec((1000, D), lambda i: (i, 0)),
                  pl.BlockSpec((1, D), lambda i: (0, 0)),
                  pl.BlockSpec((1, D), lambda i: (0, 0))],
        out_specs=pl.BlockSpec((1000, D), lambda i: (i, 0)),
        out_shape=jax.ShapeDtypeStruct((E, D), jnp.float32),
    )(stats, e_pre, w4e, e_feat, gamma_e, beta_e)


def _nstats_kernel(na_ref, dn_ref, w2_ref, st_ref):
    s = na_ref[...] / jnp.maximum(dn_ref[...], 1e-30) + w2_ref[...]
    ps = jnp.sum(s, axis=0)
    pq = jnp.sum(s * s, axis=0)
    z = jnp.zeros((6, D), jnp.float32)
    st_ref[...] = jnp.concatenate([ps[None], pq[None], z], axis=0)


def _node_stats(nacc, denom, w2h):
    return pl.pallas_call(
        _nstats_kernel,
        grid=(10,),
        in_specs=[pl.BlockSpec((1000, D), lambda i: (i, 0)),
                  pl.BlockSpec((1000, D), lambda i: (i, 0)),
                  pl.BlockSpec((1000, D), lambda i: (i, 0))],
        out_specs=pl.BlockSpec((8, D), lambda i: (i, 0)),
        out_shape=jax.ShapeDtypeStruct((10 * 8, D), jnp.float32),
    )(nacc, denom, w2h)


def _newh_kernel(st_ref, na_ref, dn_ref, w2_ref, nf_ref, g_ref, b_ref, o_ref):
    st = jnp.sum(st_ref[...].reshape(10, 8, D), axis=0)
    mean = st[0:1] / N
    var = st[1:2] / N - mean * mean
    inv = lax.rsqrt(var + EPS)
    s = na_ref[...] / jnp.maximum(dn_ref[...], 1e-30) + w2_ref[...]
    xn = (s - mean) * inv * g_ref[...] + b_ref[...]
    o_ref[...] = jnp.maximum(xn, 0.0) + nf_ref[...]


def _node_update(nacc, denom, w2h, n_feat, gamma_n, beta_n):
    stats = _node_stats(nacc, denom, w2h)
    return pl.pallas_call(
        _newh_kernel,
        grid=(10,),
        in_specs=[pl.BlockSpec((10 * 8, D), lambda i: (0, 0)),
                  pl.BlockSpec((1000, D), lambda i: (i, 0)),
                  pl.BlockSpec((1000, D), lambda i: (i, 0)),
                  pl.BlockSpec((1000, D), lambda i: (i, 0)),
                  pl.BlockSpec((1000, D), lambda i: (i, 0)),
                  pl.BlockSpec((1, D), lambda i: (0, 0)),
                  pl.BlockSpec((1, D), lambda i: (0, 0))],
        out_specs=pl.BlockSpec((1000, D), lambda i: (i, 0)),
        out_shape=jax.ShapeDtypeStruct((N, D), jnp.float32),
    )(stats, nacc, denom, w2h, n_feat, gamma_n, beta_n)


_MESH = plsc.VectorSubcoreMesh(core_axis_name="c", subcore_axis_name="s")

PAIRS = CHUNKS // 2


def _copy_idx(dst_ref, src_ref):
    for v in range(C // 16):
        sl = pl.ds(v * 16, 16)
        dst_ref[sl] = src_ref[sl]


class _EdgePipe:
    def __init__(self, c, s, sadj, dadj, dstr, ta_hbm, tb_hbm, ex_hbm,
                 sa, da, dv, dv2, b0, b1, be, isem, dsem, osem, acc):
        self.c, self.s = c, s
        self.sadj, self.dadj, self.dstr = sadj, dadj, dstr
        self.ta, self.tb, self.ex = ta_hbm, tb_hbm, ex_hbm
        self.sa, self.da, self.dv, self.dv2 = sa, da, dv, dv2
        self.b0, self.b1, self.be = b0, b1, be
        self.isem, self.dsem, self.osem = isem, dsem, osem
        self.acc = acc

    def _e0(self, k):
        return self.s * EDGES_PER_TILE + k * C

    def _i_descs(self, k, b):
        e0 = self._e0(k)
        ge = self.c * EP + e0
        return [
            (self.sadj.at[pl.ds(ge, C)], self.sa.at[b], self.isem.at[b, 0]),
            (self.dadj.at[pl.ds(ge, C)], self.da.at[b], self.isem.at[b, 1]),
            (self.dstr.at[pl.ds(e0, C)], self.dv.at[b], self.isem.at[b, 2]),
        ]

    def _d_descs(self, k, b):
        e0 = jnp.minimum(self._e0(k), E - C)
        half = self.c * H
        return [
            (self.ta.at[self.sa.at[b]], self.b0.at[b], self.dsem.at[b, 0]),
            (self.tb.at[self.da.at[b]], self.b1.at[b], self.dsem.at[b, 1]),
            (self.ex.at[pl.ds(e0, C), pl.ds(half, H)], self.be.at[b],
             self.dsem.at[b, 2]),
        ]

    def i_start(self, k, b):
        for sd in self._i_descs(k, b):
            pltpu.async_copy(*sd)

    def i_wait(self, k, b):
        for sd in self._i_descs(k, b):
            pltpu.make_async_copy(*sd).wait()

    def d_start(self, k, b):
        for sd in self._d_descs(k, b):
            pltpu.async_copy(*sd)

    def d_wait(self, k, b):
        for sd in self._d_descs(k, b):
            pltpu.make_async_copy(*sd).wait()

    def run(self):
        self.i_start(0, 0)

        def pair(g, _):
            k0 = 2 * g
            self.i_wait(k0, 0)

            @pl.when(g >= 1)
            def _():
                self.o_wait(k0 - 2, 0)
            self.d_start(k0, 0)

            @pl.when(g >= 1)
            def _():
                self.d_wait(k0 - 1, 1)
                self.comp(1)
                self.o_go(k0 - 1, 1)
            self.i_start(k0 + 1, 1)

            k1 = k0 + 1
            self.i_wait(k1, 1)

            @pl.when(g >= 1)
            def _():
                self.o_wait(k1 - 2, 1)
            self.d_start(k1, 1)

            self.d_wait(k0, 0)
            self.comp(0)
            self.o_go(k0, 0)

            @pl.when(k1 + 1 < CHUNKS)
            def _():
                self.i_start(k1 + 1, 0)
            return 0

        lax.fori_loop(0, PAIRS, pair, 0)
        self.d_wait(CHUNKS - 1, 1)
        self.comp(1)
        self.o_go(CHUNKS - 1, 1)
        self.o_wait(CHUNKS - 2, 0)
        self.o_wait(CHUNKS - 1, 1)


class _S1Pipe(_EdgePipe):
    def comp(self, b):
        _copy_idx(self.dv2.at[b], self.dv.at[b])

        @plsc.parallel_loop(0, C, unroll=2)
        def _(r):
            for j in range(H // 16):
                sl = pl.ds(j * 16, 16)
                self.b0[b, r, sl] = self.b0[b, r, sl] + self.b1[b, r, sl]
                self.be[b, r, sl] = jnp.exp(self.be[b, r, sl])

    def o_go(self, k, b):
        e0 = self._e0(k)
        half = self.c * H
        pltpu.async_copy(self.b0.at[b],
                         self.epre.at[pl.ds(e0, C), pl.ds(half, H)],
                         self.osem.at[b, 0])
        pltpu.async_copy(self.be.at[b], self.acc.at[self.dv2.at[b]],
                         self.osem.at[b, 1], add=True)

    def o_wait(self, k, b):
        e0 = self._e0(k)
        half = self.c * H
        pltpu.make_async_copy(self.b0.at[b],
                              self.epre.at[pl.ds(e0, C), pl.ds(half, H)],
                              self.osem.at[b, 0]).wait()
        pltpu.make_async_copy(self.be.at[b], self.acc.at[self.dv2.at[b]],
                              self.osem.at[b, 1]).wait()


class _S2Pipe(_EdgePipe):
    def _i_descs(self, k, b):
        e0 = self._e0(k)
        ge = self.c * EP + e0
        return [
            (self.sadj.at[pl.ds(ge, C)], self.sa.at[b], self.isem.at[b, 0]),
            (self.dstr.at[pl.ds(e0, C)], self.dv.at[b], self.isem.at[b, 1]),
        ]

    def _d_descs(self, k, b):
        e0 = jnp.minimum(self._e0(k), E - C)
        half = self.c * H
        return [
            (self.ta.at[self.sa.at[b]], self.b0.at[b], self.dsem.at[b, 0]),
            (self.ex.at[pl.ds(e0, C), pl.ds(half, H)], self.be.at[b],
             self.dsem.at[b, 1]),
        ]

    def comp(self, b):
        _copy_idx(self.dv2.at[b], self.dv.at[b])

        @plsc.parallel_loop(0, C, unroll=2)
        def _(r):
            for j in range(H // 16):
                sl = pl.ds(j * 16, 16)
                self.b0[b, r, sl] = self.b0[b, r, sl] * jnp.exp(self.be[b, r, sl])

    def o_go(self, k, b):
        pltpu.async_copy(self.b0.at[b], self.acc.at[self.dv2.at[b]],
                         self.osem.at[b, 0], add=True)

    def o_wait(self, k, b):
        pltpu.make_async_copy(self.b0.at[b], self.acc.at[self.dv2.at[b]],
                              self.osem.at[b, 0]).wait()


_SC_SCRATCH = [
    pltpu.VMEM_SHARED((NACC, H), jnp.float32),
    pltpu.VMEM((2, C), jnp.int32),
    pltpu.VMEM((2, C), jnp.int32),
    pltpu.VMEM((2, C), jnp.int32),
    pltpu.VMEM((2, C), jnp.int32),
    pltpu.VMEM((2, C, H), jnp.float32),
    pltpu.VMEM((2, C, H), jnp.float32),
    pltpu.VMEM((2, C, H), jnp.float32),
    pltpu.SemaphoreType.DMA((2, 3)),
    pltpu.SemaphoreType.DMA((2, 3)),
    pltpu.SemaphoreType.DMA((2, 2)),
]


def _sc_edge_kernel(sadj_hbm, dadj_hbm, dst_hbm, w0_hbm, w1_hbm, ex_hbm,
                    zero_hbm, epre_hbm, denom_hbm,
                    acc, sa, da, dv, dv2, b0, b1, be, isem, dsem, osem):
    c = lax.axis_index("c")
    s = lax.axis_index("s")

    @pl.when(s == 0)
    def _():
        pltpu.sync_copy(zero_hbm, acc)
    plsc.subcore_barrier()

    p = _S1Pipe(c, s, sadj_hbm, dadj_hbm, dst_hbm, w0_hbm, w1_hbm, ex_hbm,
                sa, da, dv, dv2, b0, b1, be, isem, dsem, osem, acc)
    p.epre = epre_hbm
    p.run()

    plsc.subcore_barrier()

    @pl.when(s == 0)
    def _():
        pltpu.sync_copy(acc, denom_hbm.at[c])


def _sc_edge(sadj, dadj, dst, w0cat, w1cat, ex, zeros):
    return pl.kernel(
        _sc_edge_kernel,
        out_type=[jax.ShapeDtypeStruct((EP, D), jnp.float32),
                  jax.ShapeDtypeStruct((2, NACC, H), jnp.float32)],
        mesh=_MESH,
        scratch_types=_SC_SCRATCH,
    )(sadj, dadj, dst, w0cat, w1cat, ex, zeros)


def _sc_node_kernel(sadj_hbm, dst_hbm, w3_hbm, ex_hbm, zero_hbm, ntmp_hbm,
                    acc, sa, dv, dv2, b0, be, isem, dsem, osem):
    c = lax.axis_index("c")
    s = lax.axis_index("s")

    @pl.when(s == 0)
    def _():
        pltpu.sync_copy(zero_hbm, acc)
    plsc.subcore_barrier()

    p = _S2Pipe(c, s, sadj_hbm, sadj_hbm, dst_hbm, w3_hbm, w3_hbm, ex_hbm,
                sa, sa, dv, dv2, b0, b0, be, isem, dsem, osem, acc)
    p.run()

    plsc.subcore_barrier()

    @pl.when(s == 0)
    def _():
        pltpu.sync_copy(acc, ntmp_hbm.at[c])


def _sc_node(sadj, dst, w3cat, ex, zeros):
    return pl.kernel(
        _sc_node_kernel,
        out_type=jax.ShapeDtypeStruct((2, NACC, H), jnp.float32),
        mesh=_MESH,
        scratch_types=[
            pltpu.VMEM_SHARED((NACC, H), jnp.float32),
            pltpu.VMEM((2, C), jnp.int32),
            pltpu.VMEM((2, C), jnp.int32),
            pltpu.VMEM((2, C), jnp.int32),
            pltpu.VMEM((2, C, H), jnp.float32),
            pltpu.VMEM((2, C, H), jnp.float32),
            pltpu.SemaphoreType.DMA((2, 2)),
            pltpu.SemaphoreType.DMA((2, 2)),
            pltpu.SemaphoreType.DMA((2, 1)),
        ],
    )(sadj, dst, w3cat, ex, zeros)


def _halves_cat(x):
    return jnp.concatenate([x[:, :H], x[:, H:]], axis=0)


def kernel(n_feat, e_feat, edge_index, W0, W1, W2, W3, W4,
           gamma_e, beta_e, gamma_n, beta_n):
    src = edge_index[0]
    dst = edge_index[1]
    zp = jnp.zeros((P,), jnp.int32)
    sadj = jnp.concatenate([src, zp, src + N, zp + N])
    dadj = jnp.concatenate([dst, zp, dst + N, zp + N])
    dstp = jnp.concatenate([dst, zp + N])

    wnt = jnp.concatenate([W0, W1, W2, W3], axis=0).T
    hcat = _node_matmuls(n_feat, wnt)
    w0h, w1h, w2h, w3h = (hcat[:, :D], hcat[:, D:2 * D],
                          hcat[:, 2 * D:3 * D], hcat[:, 3 * D:])

    zeros = jnp.zeros((NACC, H), jnp.float32)
    e_pre, denom = _sc_edge(sadj, dadj, dstp, _halves_cat(w0h),
                            _halves_cat(w1h), e_feat, zeros)

    w4e = _edge_mm(e_feat, W4.T)

    ntmp_h = _sc_node(sadj, dstp, _halves_cat(w3h), e_feat, zeros)

    stats = _edge_stats(e_pre, w4e)
    new_e = _edge_update(stats, e_pre, w4e, e_feat,
                         gamma_e.reshape(1, D), beta_e.reshape(1, D))

    nacc = ntmp_h[:, :N, :].transpose(1, 0, 2).reshape(N, D)
    dn = denom[:, :N, :].transpose(1, 0, 2).reshape(N, D)
    beta_n2 = (beta_n + new_e[0, :1] * 0.0).reshape(1, D)
    new_h = _node_update(nacc, dn, w2h, n_feat,
                         gamma_n.reshape(1, D), beta_n2)
    return (new_h, new_e)


# speedup vs baseline: 1.2117x; 1.0228x over previous
import functools

import jax
import jax.numpy as jnp
from jax import lax
from jax.experimental import pallas as pl
from jax.experimental.pallas import tpu as pltpu
from jax.experimental.pallas import tpu_sc as plsc

N = 10000
E = 160000
D = 256
H = 128
EPS = 1e-5

C = 64
TILES = 16
EDGES_PER_TILE = 10112
EP = EDGES_PER_TILE * TILES
P = EP - E
CHUNKS = EDGES_PER_TILE // C
NACC = N + 16


def _mm_kernel(x_ref, w_ref, o_ref):
    o_ref[...] = jnp.dot(x_ref[...], w_ref[...],
                         preferred_element_type=jnp.float32)


def _node_matmuls(n_feat, wnt):
    return pl.pallas_call(
        _mm_kernel,
        grid=(25,),
        in_specs=[pl.BlockSpec((400, D), lambda i: (i, 0)),
                  pl.BlockSpec((D, 4 * D), lambda i: (0, 0))],
        out_specs=pl.BlockSpec((400, 4 * D), lambda i: (i, 0)),
        out_shape=jax.ShapeDtypeStruct((N, 4 * D), jnp.float32),
    )(n_feat, wnt)


def _bf16_mm_kernel(x_ref, w_ref, o_ref):
    o_ref[...] = jnp.dot(x_ref[...].astype(jnp.bfloat16), w_ref[...],
                         preferred_element_type=jnp.float32)


def _edge_mm(e_feat, w4t):
    return pl.pallas_call(
        _bf16_mm_kernel,
        grid=(160,),
        in_specs=[pl.BlockSpec((1000, D), lambda i: (i, 0)),
                  pl.BlockSpec((D, D), lambda i: (0, 0))],
        out_specs=pl.BlockSpec((1000, D), lambda i: (i, 0)),
        out_shape=jax.ShapeDtypeStruct((E, D), jnp.float32),
    )(e_feat, w4t.astype(jnp.bfloat16))


def _estats_kernel(ep_ref, w4_ref, st_ref, s_ref):
    s = ep_ref[...] + w4_ref[...]
    ps = jnp.sum(s, axis=0)
    pq = jnp.sum(s * s, axis=0)
    z = jnp.zeros((6, D), jnp.float32)
    st_ref[...] = jnp.concatenate([ps[None], pq[None], z], axis=0)
    s_ref[...] = s.astype(jnp.bfloat16)


def _edge_stats(e_pre, w4e):
    return pl.pallas_call(
        _estats_kernel,
        grid=(160,),
        in_specs=[pl.BlockSpec((1000, D), lambda i: (i, 0)),
                  pl.BlockSpec((1000, D), lambda i: (i, 0))],
        out_specs=[pl.BlockSpec((8, D), lambda i: (i, 0)),
                   pl.BlockSpec((1000, D), lambda i: (i, 0))],
        out_shape=[jax.ShapeDtypeStruct((160 * 8, D), jnp.float32),
                   jax.ShapeDtypeStruct((E, D), jnp.bfloat16)],
    )(e_pre, w4e)


def _newe_kernel(st_ref, s_ref, ef_ref, g_ref, b_ref, o_ref):
    st = jnp.sum(st_ref[...].reshape(160, 8, D), axis=0)
    mean = st[0:1] / E
    var = st[1:2] / E - mean * mean
    inv = lax.rsqrt(var + EPS)
    s = s_ref[...].astype(jnp.float32)
    xn = (s - mean) * inv * g_ref[...] + b_ref[...]
    o_ref[...] = jnp.maximum(xn, 0.0) + ef_ref[...]


def _edge_update(stats, s_bf16, e_feat, gamma_e, beta_e):
    return pl.pallas_call(
        _newe_kernel,
        grid=(160,),
        in_specs=[pl.BlockSpec((160 * 8, D), lambda i: (0, 0)),
                  pl.BlockSpec((1000, D), lambda i: (i, 0)),
                  pl.BlockSpec((1000, D), lambda i: (i, 0)),
                  pl.BlockSpec((1, D), lambda i: (0, 0)),
                  pl.BlockSpec((1, D), lambda i: (0, 0))],
        out_specs=pl.BlockSpec((1000, D), lambda i: (i, 0)),
        out_shape=jax.ShapeDtypeStruct((E, D), jnp.float32),
    )(stats, s_bf16, e_feat, gamma_e, beta_e)


def _nstats_kernel(na_ref, dn_ref, w2_ref, st_ref):
    s = na_ref[...] / jnp.maximum(dn_ref[...], 1e-30) + w2_ref[...]
    ps = jnp.sum(s, axis=0)
    pq = jnp.sum(s * s, axis=0)
    z = jnp.zeros((6, D), jnp.float32)
    st_ref[...] = jnp.concatenate([ps[None], pq[None], z], axis=0)


def _node_stats(nacc, denom, w2h):
    return pl.pallas_call(
        _nstats_kernel,
        grid=(10,),
        in_specs=[pl.BlockSpec((1000, D), lambda i: (i, 0)),
                  pl.BlockSpec((1000, D), lambda i: (i, 0)),
                  pl.BlockSpec((1000, D), lambda i: (i, 0))],
        out_specs=pl.BlockSpec((8, D), lambda i: (i, 0)),
        out_shape=jax.ShapeDtypeStruct((10 * 8, D), jnp.float32),
    )(nacc, denom, w2h)


def _newh_kernel(st_ref, na_ref, dn_ref, w2_ref, nf_ref, g_ref, b_ref, o_ref):
    st = jnp.sum(st_ref[...].reshape(10, 8, D), axis=0)
    mean = st[0:1] / N
    var = st[1:2] / N - mean * mean
    inv = lax.rsqrt(var + EPS)
    s = na_ref[...] / jnp.maximum(dn_ref[...], 1e-30) + w2_ref[...]
    xn = (s - mean) * inv * g_ref[...] + b_ref[...]
    o_ref[...] = jnp.maximum(xn, 0.0) + nf_ref[...]


def _node_update(nacc, denom, w2h, n_feat, gamma_n, beta_n):
    stats = _node_stats(nacc, denom, w2h)
    return pl.pallas_call(
        _newh_kernel,
        grid=(10,),
        in_specs=[pl.BlockSpec((10 * 8, D), lambda i: (0, 0)),
                  pl.BlockSpec((1000, D), lambda i: (i, 0)),
                  pl.BlockSpec((1000, D), lambda i: (i, 0)),
                  pl.BlockSpec((1000, D), lambda i: (i, 0)),
                  pl.BlockSpec((1000, D), lambda i: (i, 0)),
                  pl.BlockSpec((1, D), lambda i: (0, 0)),
                  pl.BlockSpec((1, D), lambda i: (0, 0))],
        out_specs=pl.BlockSpec((1000, D), lambda i: (i, 0)),
        out_shape=jax.ShapeDtypeStruct((N, D), jnp.float32),
    )(stats, nacc, denom, w2h, n_feat, gamma_n, beta_n)


_MESH = plsc.VectorSubcoreMesh(core_axis_name="c", subcore_axis_name="s")

PAIRS = CHUNKS // 2


def _sub_idx(dst_ref, src_ref, off):
    for v in range(C // 16):
        sl = pl.ds(v * 16, 16)
        dst_ref[sl] = src_ref[sl] - off


class _EdgePipe:
    def __init__(self, c, s, sadj, dadj, dstr, ta_hbm, tb_hbm, ex_hbm,
                 sa, da, dv, dv2, b0, b1, be, isem, dsem, osem, acc):
        self.c, self.s = c, s
        self.sadj, self.dadj, self.dstr = sadj, dadj, dstr
        self.ta, self.tb, self.ex = ta_hbm, tb_hbm, ex_hbm
        self.sa, self.da, self.dv, self.dv2 = sa, da, dv, dv2
        self.b0, self.b1, self.be = b0, b1, be
        self.isem, self.dsem, self.osem = isem, dsem, osem
        self.acc = acc

    def _e0(self, k):
        return self.s * EDGES_PER_TILE + k * C

    def _i_descs(self, k, b):
        e0 = self._e0(k)
        ge = self.c * EP + e0
        return [
            (self.sadj.at[pl.ds(ge, C)], self.sa.at[b], self.isem.at[b, 0]),
            (self.dadj.at[pl.ds(ge, C)], self.da.at[b], self.isem.at[b, 1]),
        ]

    def _d_descs(self, k, b):
        e0 = jnp.minimum(self._e0(k), E - C)
        half = self.c * H
        return [
            (self.ta.at[self.sa.at[b]], self.b0.at[b], self.dsem.at[b, 0]),
            (self.tb.at[self.da.at[b]], self.b1.at[b], self.dsem.at[b, 1]),
            (self.ex.at[pl.ds(e0, C), pl.ds(half, H)], self.be.at[b],
             self.dsem.at[b, 2]),
        ]

    def i_start(self, k, b):
        for sd in self._i_descs(k, b):
            pltpu.async_copy(*sd)

    def i_wait(self, k, b):
        for sd in self._i_descs(k, b):
            pltpu.make_async_copy(*sd).wait()

    def d_start(self, k, b):
        for sd in self._d_descs(k, b):
            pltpu.async_copy(*sd)

    def d_wait(self, k, b):
        for sd in self._d_descs(k, b):
            pltpu.make_async_copy(*sd).wait()

    def run(self):
        self.i_start(0, 0)

        def pair(g, _):
            k0 = 2 * g
            self.i_wait(k0, 0)

            @pl.when(g >= 1)
            def _():
                self.o_wait(k0 - 2, 0)
            self.d_start(k0, 0)

            @pl.when(g >= 1)
            def _():
                self.d_wait(k0 - 1, 1)
                self.comp(1)
                self.o_go(k0 - 1, 1)
            self.i_start(k0 + 1, 1)

            k1 = k0 + 1
            self.i_wait(k1, 1)

            @pl.when(g >= 1)
            def _():
                self.o_wait(k1 - 2, 1)
            self.d_start(k1, 1)

            self.d_wait(k0, 0)
            self.comp(0)
            self.o_go(k0, 0)

            @pl.when(k1 + 1 < CHUNKS)
            def _():
                self.i_start(k1 + 1, 0)
            return 0

        lax.fori_loop(0, PAIRS, pair, 0)
        self.d_wait(CHUNKS - 1, 1)
        self.comp(1)
        self.o_go(CHUNKS - 1, 1)
        self.o_wait(CHUNKS - 2, 0)
        self.o_wait(CHUNKS - 1, 1)


class _S1Pipe(_EdgePipe):
    def comp(self, b):
        _sub_idx(self.dv2.at[b], self.da.at[b], self.c * N)

        @plsc.parallel_loop(0, C, unroll=4)
        def _(r):
            for j in range(H // 16):
                sl = pl.ds(j * 16, 16)
                self.b0[b, r, sl] = self.b0[b, r, sl] + self.b1[b, r, sl]
                self.be[b, r, sl] = jnp.exp(self.be[b, r, sl])

    def o_go(self, k, b):
        e0 = self._e0(k)
        half = self.c * H
        pltpu.async_copy(self.b0.at[b],
                         self.epre.at[pl.ds(e0, C), pl.ds(half, H)],
                         self.osem.at[b, 0])
        pltpu.async_copy(self.be.at[b], self.acc.at[self.dv2.at[b]],
                         self.osem.at[b, 1], add=True)

    def o_wait(self, k, b):
        e0 = self._e0(k)
        half = self.c * H
        pltpu.make_async_copy(self.b0.at[b],
                              self.epre.at[pl.ds(e0, C), pl.ds(half, H)],
                              self.osem.at[b, 0]).wait()
        pltpu.make_async_copy(self.be.at[b], self.acc.at[self.dv2.at[b]],
                              self.osem.at[b, 1]).wait()


class _S2Pipe(_EdgePipe):
    def _d_descs(self, k, b):
        e0 = jnp.minimum(self._e0(k), E - C)
        half = self.c * H
        return [
            (self.ta.at[self.sa.at[b]], self.b0.at[b], self.dsem.at[b, 0]),
            (self.ex.at[pl.ds(e0, C), pl.ds(half, H)], self.be.at[b],
             self.dsem.at[b, 1]),
        ]

    def comp(self, b):
        _sub_idx(self.dv2.at[b], self.da.at[b], self.c * N)

        @plsc.parallel_loop(0, C, unroll=4)
        def _(r):
            for j in range(H // 16):
                sl = pl.ds(j * 16, 16)
                self.b0[b, r, sl] = self.b0[b, r, sl] * jnp.exp(self.be[b, r, sl])

    def o_go(self, k, b):
        pltpu.async_copy(self.b0.at[b], self.acc.at[self.dv2.at[b]],
                         self.osem.at[b, 0], add=True)

    def o_wait(self, k, b):
        pltpu.make_async_copy(self.b0.at[b], self.acc.at[self.dv2.at[b]],
                              self.osem.at[b, 0]).wait()


def _sc_edge_kernel(sadj_hbm, dadj_hbm, w0_hbm, w1_hbm, ex_hbm,
                    zero_hbm, epre_hbm, denom_hbm,
                    acc, sa, da, dv2, b0, b1, be, isem, dsem, osem):
    c = lax.axis_index("c")
    s = lax.axis_index("s")

    @pl.when(s == 0)
    def _():
        pltpu.sync_copy(zero_hbm, acc)
    plsc.subcore_barrier()

    p = _S1Pipe(c, s, sadj_hbm, dadj_hbm, None, w0_hbm, w1_hbm, ex_hbm,
                sa, da, None, dv2, b0, b1, be, isem, dsem, osem, acc)
    p.epre = epre_hbm
    p.run()

    plsc.subcore_barrier()

    @pl.when(s == 0)
    def _():
        pltpu.sync_copy(acc, denom_hbm.at[c])


def _sc_edge(sadj, dadj, w0cat, w1cat, ex, zeros):
    return pl.kernel(
        _sc_edge_kernel,
        out_type=[jax.ShapeDtypeStruct((EP, D), jnp.float32),
                  jax.ShapeDtypeStruct((2, NACC, H), jnp.float32)],
        mesh=_MESH,
        scratch_types=[
            pltpu.VMEM_SHARED((NACC, H), jnp.float32),
            pltpu.VMEM((2, C), jnp.int32),
            pltpu.VMEM((2, C), jnp.int32),
            pltpu.VMEM((2, C), jnp.int32),
            pltpu.VMEM((2, C, H), jnp.float32),
            pltpu.VMEM((2, C, H), jnp.float32),
            pltpu.VMEM((2, C, H), jnp.float32),
            pltpu.SemaphoreType.DMA((2, 2)),
            pltpu.SemaphoreType.DMA((2, 3)),
            pltpu.SemaphoreType.DMA((2, 2)),
        ],
    )(sadj, dadj, w0cat, w1cat, ex, zeros)


def _sc_node_kernel(sadj_hbm, dadj_hbm, w3_hbm, ex_hbm, zero_hbm, ntmp_hbm,
                    acc, sa, da, dv2, b0, be, isem, dsem, osem):
    c = lax.axis_index("c")
    s = lax.axis_index("s")

    @pl.when(s == 0)
    def _():
        pltpu.sync_copy(zero_hbm, acc)
    plsc.subcore_barrier()

    p = _S2Pipe(c, s, sadj_hbm, dadj_hbm, None, w3_hbm, None, ex_hbm,
                sa, da, None, dv2, b0, None, be, isem, dsem, osem, acc)
    p.run()

    plsc.subcore_barrier()

    @pl.when(s == 0)
    def _():
        pltpu.sync_copy(acc, ntmp_hbm.at[c])


def _sc_node(sadj, dadj, w3cat, ex, zeros):
    return pl.kernel(
        _sc_node_kernel,
        out_type=jax.ShapeDtypeStruct((2, NACC, H), jnp.float32),
        mesh=_MESH,
        scratch_types=[
            pltpu.VMEM_SHARED((NACC, H), jnp.float32),
            pltpu.VMEM((2, C), jnp.int32),
            pltpu.VMEM((2, C), jnp.int32),
            pltpu.VMEM((2, C), jnp.int32),
            pltpu.VMEM((2, C, H), jnp.float32),
            pltpu.VMEM((2, C, H), jnp.float32),
            pltpu.SemaphoreType.DMA((2, 2)),
            pltpu.SemaphoreType.DMA((2, 2)),
            pltpu.SemaphoreType.DMA((2, 1)),
        ],
    )(sadj, dadj, w3cat, ex, zeros)


def _halves_cat(x):
    return jnp.concatenate([x[:, :H], x[:, H:]], axis=0)


def kernel(n_feat, e_feat, edge_index, W0, W1, W2, W3, W4,
           gamma_e, beta_e, gamma_n, beta_n):
    src = edge_index[0]
    dst = edge_index[1]
    zp = jnp.zeros((P,), jnp.int32)
    sadj = jnp.concatenate([src, zp, src + N, zp + N])
    dadj = jnp.concatenate([dst, zp + N, dst + N, zp + 2 * N])

    wnt = jnp.concatenate([W0, W1, W2, W3], axis=0).T
    hcat = _node_matmuls(n_feat, wnt)
    w0h, w1h, w2h, w3h = (hcat[:, :D], hcat[:, D:2 * D],
                          hcat[:, 2 * D:3 * D], hcat[:, 3 * D:])

    zeros = jnp.zeros((NACC, H), jnp.float32)
    w1cat = jnp.concatenate([_halves_cat(w1h), jnp.zeros((16, H))], axis=0)
    e_pre, denom = _sc_edge(sadj, dadj, _halves_cat(w0h),
                            w1cat, e_feat, zeros)

    w4e = _edge_mm(e_feat, W4.T)

    ntmp_h = _sc_node(sadj, dadj, _halves_cat(w3h), e_feat, zeros)

    stats, s_bf16 = _edge_stats(e_pre, w4e)
    new_e = _edge_update(stats, s_bf16, e_feat,
                         gamma_e.reshape(1, D), beta_e.reshape(1, D))

    nacc = ntmp_h[:, :N, :].transpose(1, 0, 2).reshape(N, D)
    dn = denom[:, :N, :].transpose(1, 0, 2).reshape(N, D)
    beta_n2 = (beta_n + new_e[0, :1] * 0.0).reshape(1, D)
    new_h = _node_update(nacc, dn, w2h, n_feat,
                         gamma_n.reshape(1, D), beta_n2)
    return (new_h, new_e)


# speedup vs baseline: 1.2327x; 1.0173x over previous
import functools

import jax
import jax.numpy as jnp
from jax import lax
from jax.experimental import pallas as pl
from jax.experimental.pallas import tpu as pltpu
from jax.experimental.pallas import tpu_sc as plsc

N = 10000
E = 160000
D = 256
H = 128
EPS = 1e-5

C = 64
TILES = 16
EDGES_PER_TILE = 10112
EP = EDGES_PER_TILE * TILES
P = EP - E
CHUNKS = EDGES_PER_TILE // C
NACC = N + 16


def _mm_kernel(x_ref, w_ref, o_ref):
    o_ref[...] = jnp.dot(x_ref[...], w_ref[...],
                         preferred_element_type=jnp.float32)


def _node_matmuls(n_feat, wnt):
    return pl.pallas_call(
        _mm_kernel,
        grid=(25,),
        in_specs=[pl.BlockSpec((400, D), lambda i: (i, 0)),
                  pl.BlockSpec((D, 4 * D), lambda i: (0, 0))],
        out_specs=pl.BlockSpec((400, 4 * D), lambda i: (i, 0)),
        out_shape=jax.ShapeDtypeStruct((N, 4 * D), jnp.float32),
    )(n_feat, wnt)


def _bf16_mm_kernel(x_ref, w_ref, o_ref):
    o_ref[...] = jnp.dot(x_ref[...].astype(jnp.bfloat16), w_ref[...],
                         preferred_element_type=jnp.float32)


def _edge_mm(e_feat, w4t):
    return pl.pallas_call(
        _bf16_mm_kernel,
        grid=(160,),
        in_specs=[pl.BlockSpec((1000, D), lambda i: (i, 0)),
                  pl.BlockSpec((D, D), lambda i: (0, 0))],
        out_specs=pl.BlockSpec((1000, D), lambda i: (i, 0)),
        out_shape=jax.ShapeDtypeStruct((E, D), jnp.float32),
    )(e_feat, w4t.astype(jnp.bfloat16))


def _estats_kernel(ep_ref, w4_ref, st_ref, s_ref):
    s = ep_ref[...] + w4_ref[...]
    ps = jnp.sum(s, axis=0)
    pq = jnp.sum(s * s, axis=0)
    z = jnp.zeros((6, D), jnp.float32)
    st_ref[...] = jnp.concatenate([ps[None], pq[None], z], axis=0)
    s_ref[...] = s.astype(jnp.bfloat16)


def _edge_stats(e_pre, w4e):
    return pl.pallas_call(
        _estats_kernel,
        grid=(160,),
        in_specs=[pl.BlockSpec((1000, D), lambda i: (i, 0)),
                  pl.BlockSpec((1000, D), lambda i: (i, 0))],
        out_specs=[pl.BlockSpec((8, D), lambda i: (i, 0)),
                   pl.BlockSpec((1000, D), lambda i: (i, 0))],
        out_shape=[jax.ShapeDtypeStruct((160 * 8, D), jnp.float32),
                   jax.ShapeDtypeStruct((E, D), jnp.bfloat16)],
    )(e_pre, w4e)


def _newe_kernel(st_ref, s_ref, ef_ref, g_ref, b_ref, o_ref):
    st = jnp.sum(st_ref[...].reshape(160, 8, D), axis=0)
    mean = st[0:1] / E
    var = st[1:2] / E - mean * mean
    inv = lax.rsqrt(var + EPS)
    s = s_ref[...].astype(jnp.float32)
    xn = (s - mean) * inv * g_ref[...] + b_ref[...]
    o_ref[...] = jnp.maximum(xn, 0.0) + ef_ref[...]


def _edge_update(stats, s_bf16, e_feat, gamma_e, beta_e):
    return pl.pallas_call(
        _newe_kernel,
        grid=(160,),
        in_specs=[pl.BlockSpec((160 * 8, D), lambda i: (0, 0)),
                  pl.BlockSpec((1000, D), lambda i: (i, 0)),
                  pl.BlockSpec((1000, D), lambda i: (i, 0)),
                  pl.BlockSpec((1, D), lambda i: (0, 0)),
                  pl.BlockSpec((1, D), lambda i: (0, 0))],
        out_specs=pl.BlockSpec((1000, D), lambda i: (i, 0)),
        out_shape=jax.ShapeDtypeStruct((E, D), jnp.float32),
    )(stats, s_bf16, e_feat, gamma_e, beta_e)


def _nstats_kernel(na_ref, dn_ref, w2_ref, st_ref):
    s = na_ref[...] / jnp.maximum(dn_ref[...], 1e-30) + w2_ref[...]
    ps = jnp.sum(s, axis=0)
    pq = jnp.sum(s * s, axis=0)
    z = jnp.zeros((6, D), jnp.float32)
    st_ref[...] = jnp.concatenate([ps[None], pq[None], z], axis=0)


def _node_stats(nacc, denom, w2h):
    return pl.pallas_call(
        _nstats_kernel,
        grid=(10,),
        in_specs=[pl.BlockSpec((1000, D), lambda i: (i, 0)),
                  pl.BlockSpec((1000, D), lambda i: (i, 0)),
                  pl.BlockSpec((1000, D), lambda i: (i, 0))],
        out_specs=pl.BlockSpec((8, D), lambda i: (i, 0)),
        out_shape=jax.ShapeDtypeStruct((10 * 8, D), jnp.float32),
    )(nacc, denom, w2h)


def _newh_kernel(st_ref, na_ref, dn_ref, w2_ref, nf_ref, g_ref, b_ref, o_ref):
    st = jnp.sum(st_ref[...].reshape(10, 8, D), axis=0)
    mean = st[0:1] / N
    var = st[1:2] / N - mean * mean
    inv = lax.rsqrt(var + EPS)
    s = na_ref[...] / jnp.maximum(dn_ref[...], 1e-30) + w2_ref[...]
    xn = (s - mean) * inv * g_ref[...] + b_ref[...]
    o_ref[...] = jnp.maximum(xn, 0.0) + nf_ref[...]


def _node_update(nacc, denom, w2h, n_feat, gamma_n, beta_n):
    stats = _node_stats(nacc, denom, w2h)
    return pl.pallas_call(
        _newh_kernel,
        grid=(10,),
        in_specs=[pl.BlockSpec((10 * 8, D), lambda i: (0, 0)),
                  pl.BlockSpec((1000, D), lambda i: (i, 0)),
                  pl.BlockSpec((1000, D), lambda i: (i, 0)),
                  pl.BlockSpec((1000, D), lambda i: (i, 0)),
                  pl.BlockSpec((1000, D), lambda i: (i, 0)),
                  pl.BlockSpec((1, D), lambda i: (0, 0)),
                  pl.BlockSpec((1, D), lambda i: (0, 0))],
        out_specs=pl.BlockSpec((1000, D), lambda i: (i, 0)),
        out_shape=jax.ShapeDtypeStruct((N, D), jnp.float32),
    )(stats, nacc, denom, w2h, n_feat, gamma_n, beta_n)


_MESH = plsc.VectorSubcoreMesh(core_axis_name="c", subcore_axis_name="s")

PAIRS = CHUNKS // 2


def _sub_idx(dst_ref, src_ref, off):
    for v in range(C // 16):
        sl = pl.ds(v * 16, 16)
        dst_ref[sl] = src_ref[sl] - off


class _EdgePipe:
    def __init__(self, c, s, sadj, dadj, dstr, ta_hbm, tb_hbm, ex_hbm,
                 sa, da, dv, dv2, b0, b1, be, isem, dsem, osem, acc):
        self.c, self.s = c, s
        self.sadj, self.dadj, self.dstr = sadj, dadj, dstr
        self.ta, self.tb, self.ex = ta_hbm, tb_hbm, ex_hbm
        self.sa, self.da, self.dv, self.dv2 = sa, da, dv, dv2
        self.b0, self.b1, self.be = b0, b1, be
        self.isem, self.dsem, self.osem = isem, dsem, osem
        self.acc = acc

    def _e0(self, k):
        return self.s * EDGES_PER_TILE + k * C

    def _i_descs(self, k, b):
        e0 = self._e0(k)
        ge = self.c * EP + e0
        return [
            (self.sadj.at[pl.ds(ge, C)], self.sa.at[b], self.isem.at[b, 0]),
            (self.dadj.at[pl.ds(ge, C)], self.da.at[b], self.isem.at[b, 1]),
        ]

    def _d_descs(self, k, b):
        e0 = jnp.minimum(self._e0(k), E - C)
        half = self.c * H
        return [
            (self.ta.at[self.sa.at[b]], self.b0.at[b], self.dsem.at[b, 0]),
            (self.tb.at[self.da.at[b]], self.b1.at[b], self.dsem.at[b, 1]),
            (self.ex.at[pl.ds(e0, C), pl.ds(half, H)], self.be.at[b],
             self.dsem.at[b, 2]),
        ]

    def i_start(self, k, b):
        for sd in self._i_descs(k, b):
            pltpu.async_copy(*sd)

    def i_wait(self, k, b):
        for sd in self._i_descs(k, b):
            pltpu.make_async_copy(*sd).wait()

    def d_start(self, k, b):
        for sd in self._d_descs(k, b):
            pltpu.async_copy(*sd)

    def d_wait(self, k, b):
        for sd in self._d_descs(k, b):
            pltpu.make_async_copy(*sd).wait()

    def run(self):
        self.i_start(0, 0)

        def pair(g, _):
            k0 = 2 * g
            self.i_wait(k0, 0)

            @pl.when(g >= 1)
            def _():
                self.o_wait(k0 - 2, 0)
            self.d_start(k0, 0)

            @pl.when(g >= 1)
            def _():
                self.d_wait(k0 - 1, 1)
                self.comp(1)
                self.o_go(k0 - 1, 1)
            self.i_start(k0 + 1, 1)

            k1 = k0 + 1
            self.i_wait(k1, 1)

            @pl.when(g >= 1)
            def _():
                self.o_wait(k1 - 2, 1)
            self.d_start(k1, 1)

            self.d_wait(k0, 0)
            self.comp(0)
            self.o_go(k0, 0)

            @pl.when(k1 + 1 < CHUNKS)
            def _():
                self.i_start(k1 + 1, 0)
            return 0

        lax.fori_loop(0, PAIRS, pair, 0)
        self.d_wait(CHUNKS - 1, 1)
        self.comp(1)
        self.o_go(CHUNKS - 1, 1)
        self.o_wait(CHUNKS - 2, 0)
        self.o_wait(CHUNKS - 1, 1)


class _S1Pipe(_EdgePipe):
    def _i_descs(self, k, b):
        m = self.s * CHUNKS + k
        return [
            (self.sadj.at[self.c, m], self.sa.at[b], self.isem.at[b, 0]),
        ]

    def _d_descs(self, k, b):
        e0 = jnp.minimum(self._e0(k), E - C)
        half = self.c * H
        return [
            (self.ta.at[self.sa.at[b]], self.b0.at[b], self.dsem.at[b, 0]),
            (self.ex.at[pl.ds(e0, C), pl.ds(half, H)], self.be.at[b],
             self.dsem.at[b, 1]),
        ]

    def comp(self, b):
        _sub_idx(self.dv2.at[b], self.sa.at[b, pl.ds(C, C)],
                 2 * N + self.c * N)

        @plsc.parallel_loop(0, C, unroll=4)
        def _(r):
            for j in range(H // 16):
                sl = pl.ds(j * 16, 16)
                self.b0[b, r, sl] = self.b0[b, r, sl] + self.b0[b, C + r, sl]
                self.be[b, r, sl] = jnp.exp(self.be[b, r, sl])

    def o_go(self, k, b):
        e0 = self._e0(k)
        half = self.c * H
        pltpu.async_copy(self.b0.at[b, pl.ds(0, C)],
                         self.epre.at[pl.ds(e0, C), pl.ds(half, H)],
                         self.osem.at[b, 0])
        pltpu.async_copy(self.be.at[b], self.acc.at[self.dv2.at[b]],
                         self.osem.at[b, 1], add=True)

    def o_wait(self, k, b):
        e0 = self._e0(k)
        half = self.c * H
        pltpu.make_async_copy(self.b0.at[b, pl.ds(0, C)],
                              self.epre.at[pl.ds(e0, C), pl.ds(half, H)],
                              self.osem.at[b, 0]).wait()
        pltpu.make_async_copy(self.be.at[b], self.acc.at[self.dv2.at[b]],
                              self.osem.at[b, 1]).wait()


class _S2Pipe(_EdgePipe):
    def _d_descs(self, k, b):
        e0 = jnp.minimum(self._e0(k), E - C)
        half = self.c * H
        return [
            (self.ta.at[self.sa.at[b]], self.b0.at[b], self.dsem.at[b, 0]),
            (self.ex.at[pl.ds(e0, C), pl.ds(half, H)], self.be.at[b],
             self.dsem.at[b, 1]),
        ]

    def comp(self, b):
        _sub_idx(self.dv2.at[b], self.da.at[b], self.c * N)

        @plsc.parallel_loop(0, C, unroll=4)
        def _(r):
            for j in range(H // 16):
                sl = pl.ds(j * 16, 16)
                self.b0[b, r, sl] = self.b0[b, r, sl] * jnp.exp(self.be[b, r, sl])

    def o_go(self, k, b):
        pltpu.async_copy(self.b0.at[b], self.acc.at[self.dv2.at[b]],
                         self.osem.at[b, 0], add=True)

    def o_wait(self, k, b):
        pltpu.make_async_copy(self.b0.at[b], self.acc.at[self.dv2.at[b]],
                              self.osem.at[b, 0]).wait()


def _sc_edge_kernel(gidx_hbm, w01_hbm, ex_hbm, zero_hbm, epre_hbm, denom_hbm,
                    acc, si, dv2, bg, be, isem, dsem, osem):
    c = lax.axis_index("c")
    s = lax.axis_index("s")

    @pl.when(s == 0)
    def _():
        pltpu.sync_copy(zero_hbm, acc)
    plsc.subcore_barrier()

    p = _S1Pipe(c, s, gidx_hbm, None, None, w01_hbm, None, ex_hbm,
                si, None, None, dv2, bg, None, be, isem, dsem, osem, acc)
    p.epre = epre_hbm
    p.run()

    plsc.subcore_barrier()

    @pl.when(s == 0)
    def _():
        pltpu.sync_copy(acc, denom_hbm.at[c])


def _sc_edge(gidx, w01cat, ex, zeros):
    return pl.kernel(
        _sc_edge_kernel,
        out_type=[jax.ShapeDtypeStruct((EP, D), jnp.float32),
                  jax.ShapeDtypeStruct((2, NACC, H), jnp.float32)],
        mesh=_MESH,
        scratch_types=[
            pltpu.VMEM_SHARED((NACC, H), jnp.float32),
            pltpu.VMEM((2, 2 * C), jnp.int32),
            pltpu.VMEM((2, C), jnp.int32),
            pltpu.VMEM((2, 2 * C, H), jnp.float32),
            pltpu.VMEM((2, C, H), jnp.float32),
            pltpu.SemaphoreType.DMA((2, 1)),
            pltpu.SemaphoreType.DMA((2, 2)),
            pltpu.SemaphoreType.DMA((2, 2)),
        ],
    )(gidx, w01cat, ex, zeros)


def _sc_node_kernel(sadj_hbm, dadj_hbm, w3_hbm, ex_hbm, zero_hbm, ntmp_hbm,
                    acc, sa, da, dv2, b0, be, isem, dsem, osem):
    c = lax.axis_index("c")
    s = lax.axis_index("s")

    @pl.when(s == 0)
    def _():
        pltpu.sync_copy(zero_hbm, acc)
    plsc.subcore_barrier()

    p = _S2Pipe(c, s, sadj_hbm, dadj_hbm, None, w3_hbm, None, ex_hbm,
                sa, da, None, dv2, b0, None, be, isem, dsem, osem, acc)
    p.run()

    plsc.subcore_barrier()

    @pl.when(s == 0)
    def _():
        pltpu.sync_copy(acc, ntmp_hbm.at[c])


def _sc_node(sadj, dadj, w3cat, ex, zeros):
    return pl.kernel(
        _sc_node_kernel,
        out_type=jax.ShapeDtypeStruct((2, NACC, H), jnp.float32),
        mesh=_MESH,
        scratch_types=[
            pltpu.VMEM_SHARED((NACC, H), jnp.float32),
            pltpu.VMEM((2, C), jnp.int32),
            pltpu.VMEM((2, C), jnp.int32),
            pltpu.VMEM((2, C), jnp.int32),
            pltpu.VMEM((2, C, H), jnp.float32),
            pltpu.VMEM((2, C, H), jnp.float32),
            pltpu.SemaphoreType.DMA((2, 2)),
            pltpu.SemaphoreType.DMA((2, 2)),
            pltpu.SemaphoreType.DMA((2, 1)),
        ],
    )(sadj, dadj, w3cat, ex, zeros)


def _halves_cat(x):
    return jnp.concatenate([x[:, :H], x[:, H:]], axis=0)


def kernel(n_feat, e_feat, edge_index, W0, W1, W2, W3, W4,
           gamma_e, beta_e, gamma_n, beta_n):
    src = edge_index[0]
    dst = edge_index[1]
    zp = jnp.zeros((P,), jnp.int32)
    sadj = jnp.concatenate([src, zp, src + N, zp + N])
    dadj = jnp.concatenate([dst, zp + N, dst + N, zp + 2 * N])

    wnt = jnp.concatenate([W0, W1, W2, W3], axis=0).T
    hcat = _node_matmuls(n_feat, wnt)
    w0h, w1h, w2h, w3h = (hcat[:, :D], hcat[:, D:2 * D],
                          hcat[:, 2 * D:3 * D], hcat[:, 3 * D:])

    zeros = jnp.zeros((NACC, H), jnp.float32)
    w1cat = jnp.concatenate([_halves_cat(w1h), jnp.zeros((16, H))], axis=0)
    w01cat = jnp.concatenate([_halves_cat(w0h), w1cat], axis=0)
    gidx = jnp.concatenate([sadj.reshape(2, EP // C, C),
                            (dadj + 2 * N).reshape(2, EP // C, C)], axis=2)
    e_pre, denom = _sc_edge(gidx, w01cat, e_feat, zeros)

    w4e = _edge_mm(e_feat, W4.T)

    ntmp_h = _sc_node(sadj, dadj, _halves_cat(w3h), e_feat, zeros)

    stats, s_bf16 = _edge_stats(e_pre, w4e)
    new_e = _edge_update(stats, s_bf16, e_feat,
                         gamma_e.reshape(1, D), beta_e.reshape(1, D))

    nacc = ntmp_h[:, :N, :].transpose(1, 0, 2).reshape(N, D)
    dn = denom[:, :N, :].transpose(1, 0, 2).reshape(N, D)
    beta_n2 = (beta_n + new_e[0, :1] * 0.0).reshape(1, D)
    new_h = _node_update(nacc, dn, w2h, n_feat,
                         gamma_n.reshape(1, D), beta_n2)
    return (new_h, new_e)


# speedup vs baseline: 1.2902x; 1.0466x over previous
import functools

import jax
import jax.numpy as jnp
from jax import lax
from jax.experimental import pallas as pl
from jax.experimental.pallas import tpu as pltpu
from jax.experimental.pallas import tpu_sc as plsc

N = 10000
E = 160000
D = 256
H = 128
EPS = 1e-5

C = 64
TILES = 16
EDGES_PER_TILE = 10112
EP = EDGES_PER_TILE * TILES
P = EP - E
CHUNKS = EDGES_PER_TILE // C
NACC = N + 16


def _mm_kernel(x_ref, w_ref, o_ref):
    o_ref[...] = jnp.dot(x_ref[...], w_ref[...],
                         preferred_element_type=jnp.float32)


def _node_matmuls(n_feat, wnt):
    return pl.pallas_call(
        _mm_kernel,
        grid=(25,),
        in_specs=[pl.BlockSpec((400, D), lambda i: (i, 0)),
                  pl.BlockSpec((D, 4 * D), lambda i: (0, 0))],
        out_specs=pl.BlockSpec((400, 4 * D), lambda i: (i, 0)),
        out_shape=jax.ShapeDtypeStruct((N, 4 * D), jnp.float32),
    )(n_feat, wnt)


def _bf16_mm_kernel(x_ref, w_ref, o_ref):
    o_ref[...] = jnp.dot(x_ref[...].astype(jnp.bfloat16), w_ref[...],
                         preferred_element_type=jnp.float32
                         ).astype(jnp.bfloat16)


def _edge_mm(e_feat, w4t):
    return pl.pallas_call(
        _bf16_mm_kernel,
        grid=(160,),
        in_specs=[pl.BlockSpec((1000, D), lambda i: (i, 0)),
                  pl.BlockSpec((D, D), lambda i: (0, 0))],
        out_specs=pl.BlockSpec((1000, D), lambda i: (i, 0)),
        out_shape=jax.ShapeDtypeStruct((E, D), jnp.bfloat16),
    )(e_feat, w4t.astype(jnp.bfloat16))


def _estats_kernel(ep_ref, w4_ref, st_ref, s_ref):
    s = ep_ref[...] + w4_ref[...].astype(jnp.float32)
    ps = jnp.sum(s, axis=0)
    pq = jnp.sum(s * s, axis=0)
    z = jnp.zeros((6, D), jnp.float32)
    st_ref[...] = jnp.concatenate([ps[None], pq[None], z], axis=0)
    s_ref[...] = s.astype(jnp.bfloat16)


def _edge_stats(e_pre, w4e):
    return pl.pallas_call(
        _estats_kernel,
        grid=(160,),
        in_specs=[pl.BlockSpec((1000, D), lambda i: (i, 0)),
                  pl.BlockSpec((1000, D), lambda i: (i, 0))],
        out_specs=[pl.BlockSpec((8, D), lambda i: (i, 0)),
                   pl.BlockSpec((1000, D), lambda i: (i, 0))],
        out_shape=[jax.ShapeDtypeStruct((160 * 8, D), jnp.float32),
                   jax.ShapeDtypeStruct((E, D), jnp.bfloat16)],
    )(e_pre, w4e)


def _newe_kernel(st_ref, s_ref, ef_ref, g_ref, b_ref, o_ref):
    st = jnp.sum(st_ref[...].reshape(160, 8, D), axis=0)
    mean = st[0:1] / E
    var = st[1:2] / E - mean * mean
    inv = lax.rsqrt(var + EPS)
    s = s_ref[...].astype(jnp.float32)
    xn = (s - mean) * inv * g_ref[...] + b_ref[...]
    o_ref[...] = jnp.maximum(xn, 0.0) + ef_ref[...]


def _edge_update(stats, s_bf16, e_feat, gamma_e, beta_e):
    return pl.pallas_call(
        _newe_kernel,
        grid=(160,),
        in_specs=[pl.BlockSpec((160 * 8, D), lambda i: (0, 0)),
                  pl.BlockSpec((1000, D), lambda i: (i, 0)),
                  pl.BlockSpec((1000, D), lambda i: (i, 0)),
                  pl.BlockSpec((1, D), lambda i: (0, 0)),
                  pl.BlockSpec((1, D), lambda i: (0, 0))],
        out_specs=pl.BlockSpec((1000, D), lambda i: (i, 0)),
        out_shape=jax.ShapeDtypeStruct((E, D), jnp.float32),
    )(stats, s_bf16, e_feat, gamma_e, beta_e)


def _nstats_kernel(na_ref, dn_ref, w2_ref, st_ref):
    s = na_ref[...] / jnp.maximum(dn_ref[...], 1e-30) + w2_ref[...]
    ps = jnp.sum(s, axis=0)
    pq = jnp.sum(s * s, axis=0)
    z = jnp.zeros((6, D), jnp.float32)
    st_ref[...] = jnp.concatenate([ps[None], pq[None], z], axis=0)


def _node_stats(nacc, denom, w2h):
    return pl.pallas_call(
        _nstats_kernel,
        grid=(10,),
        in_specs=[pl.BlockSpec((1000, D), lambda i: (i, 0)),
                  pl.BlockSpec((1000, D), lambda i: (i, 0)),
                  pl.BlockSpec((1000, D), lambda i: (i, 0))],
        out_specs=pl.BlockSpec((8, D), lambda i: (i, 0)),
        out_shape=jax.ShapeDtypeStruct((10 * 8, D), jnp.float32),
    )(nacc, denom, w2h)


def _newh_kernel(st_ref, na_ref, dn_ref, w2_ref, nf_ref, g_ref, b_ref, o_ref):
    st = jnp.sum(st_ref[...].reshape(10, 8, D), axis=0)
    mean = st[0:1] / N
    var = st[1:2] / N - mean * mean
    inv = lax.rsqrt(var + EPS)
    s = na_ref[...] / jnp.maximum(dn_ref[...], 1e-30) + w2_ref[...]
    xn = (s - mean) * inv * g_ref[...] + b_ref[...]
    o_ref[...] = jnp.maximum(xn, 0.0) + nf_ref[...]


def _node_update(nacc, denom, w2h, n_feat, gamma_n, beta_n):
    stats = _node_stats(nacc, denom, w2h)
    return pl.pallas_call(
        _newh_kernel,
        grid=(10,),
        in_specs=[pl.BlockSpec((10 * 8, D), lambda i: (0, 0)),
                  pl.BlockSpec((1000, D), lambda i: (i, 0)),
                  pl.BlockSpec((1000, D), lambda i: (i, 0)),
                  pl.BlockSpec((1000, D), lambda i: (i, 0)),
                  pl.BlockSpec((1000, D), lambda i: (i, 0)),
                  pl.BlockSpec((1, D), lambda i: (0, 0)),
                  pl.BlockSpec((1, D), lambda i: (0, 0))],
        out_specs=pl.BlockSpec((1000, D), lambda i: (i, 0)),
        out_shape=jax.ShapeDtypeStruct((N, D), jnp.float32),
    )(stats, nacc, denom, w2h, n_feat, gamma_n, beta_n)


_MESH = plsc.VectorSubcoreMesh(core_axis_name="c", subcore_axis_name="s")

PAIRS = CHUNKS // 2


def _sub_idx(dst_ref, src_ref, off):
    for v in range(C // 16):
        sl = pl.ds(v * 16, 16)
        dst_ref[sl] = src_ref[sl] - off


class _EdgePipe:
    def __init__(self, c, s, sadj, dadj, dstr, ta_hbm, tb_hbm, ex_hbm,
                 sa, da, dv, dv2, b0, b1, be, isem, dsem, osem, acc):
        self.c, self.s = c, s
        self.sadj, self.dadj, self.dstr = sadj, dadj, dstr
        self.ta, self.tb, self.ex = ta_hbm, tb_hbm, ex_hbm
        self.sa, self.da, self.dv, self.dv2 = sa, da, dv, dv2
        self.b0, self.b1, self.be = b0, b1, be
        self.isem, self.dsem, self.osem = isem, dsem, osem
        self.acc = acc

    def _e0(self, k):
        return self.s * EDGES_PER_TILE + k * C

    def _i_descs(self, k, b):
        e0 = self._e0(k)
        ge = self.c * EP + e0
        return [
            (self.sadj.at[pl.ds(ge, C)], self.sa.at[b], self.isem.at[b, 0]),
            (self.dadj.at[pl.ds(ge, C)], self.da.at[b], self.isem.at[b, 1]),
        ]

    def _d_descs(self, k, b):
        e0 = jnp.minimum(self._e0(k), E - C)
        half = self.c * H
        return [
            (self.ta.at[self.sa.at[b]], self.b0.at[b], self.dsem.at[b, 0]),
            (self.tb.at[self.da.at[b]], self.b1.at[b], self.dsem.at[b, 1]),
            (self.ex.at[pl.ds(e0, C), pl.ds(half, H)], self.be.at[b],
             self.dsem.at[b, 2]),
        ]

    def i_start(self, k, b):
        for sd in self._i_descs(k, b):
            pltpu.async_copy(*sd)

    def i_wait(self, k, b):
        for sd in self._i_descs(k, b):
            pltpu.make_async_copy(*sd).wait()

    def d_start(self, k, b):
        for sd in self._d_descs(k, b):
            pltpu.async_copy(*sd)

    def d_wait(self, k, b):
        for sd in self._d_descs(k, b):
            pltpu.make_async_copy(*sd).wait()

    def run(self):
        self.i_start(0, 0)

        def pair(g, _):
            k0 = 2 * g
            self.i_wait(k0, 0)

            @pl.when(g >= 1)
            def _():
                self.o_wait(k0 - 2, 0)
            self.d_start(k0, 0)

            @pl.when(g >= 1)
            def _():
                self.d_wait(k0 - 1, 1)
                self.comp(1)
                self.o_go(k0 - 1, 1)
            self.i_start(k0 + 1, 1)

            k1 = k0 + 1
            self.i_wait(k1, 1)

            @pl.when(g >= 1)
            def _():
                self.o_wait(k1 - 2, 1)
            self.d_start(k1, 1)

            self.d_wait(k0, 0)
            self.comp(0)
            self.o_go(k0, 0)

            @pl.when(k1 + 1 < CHUNKS)
            def _():
                self.i_start(k1 + 1, 0)
            return 0

        lax.fori_loop(0, PAIRS, pair, 0)
        self.d_wait(CHUNKS - 1, 1)
        self.comp(1)
        self.o_go(CHUNKS - 1, 1)
        self.o_wait(CHUNKS - 2, 0)
        self.o_wait(CHUNKS - 1, 1)


class _S1Pipe(_EdgePipe):
    def _i_descs(self, k, b):
        m = self.s * CHUNKS + k
        return [
            (self.sadj.at[self.c, m], self.sa.at[b], self.isem.at[b, 0]),
        ]

    def _d_descs(self, k, b):
        e0 = jnp.minimum(self._e0(k), E - C)
        half = self.c * H
        return [
            (self.ta.at[self.sa.at[b]], self.b0.at[b], self.dsem.at[b, 0]),
            (self.ex.at[pl.ds(e0, C), pl.ds(half, H)], self.be.at[b],
             self.dsem.at[b, 1]),
        ]

    def comp(self, b):
        _sub_idx(self.dv2.at[b], self.sa.at[b, pl.ds(C, C)],
                 2 * N + self.c * N)

        @plsc.parallel_loop(0, C, unroll=4)
        def _(r):
            for j in range(H // 16):
                sl = pl.ds(j * 16, 16)
                self.b0[b, r, sl] = self.b0[b, r, sl] + self.b0[b, C + r, sl]
                self.be[b, r, sl] = jnp.exp(self.be[b, r, sl])

    def o_go(self, k, b):
        e0 = self._e0(k)
        half = self.c * H
        pltpu.async_copy(self.b0.at[b, pl.ds(0, C)],
                         self.epre.at[pl.ds(e0, C), pl.ds(half, H)],
                         self.osem.at[b, 0])
        pltpu.async_copy(self.be.at[b], self.acc.at[self.dv2.at[b]],
                         self.osem.at[b, 1], add=True)

    def o_wait(self, k, b):
        e0 = self._e0(k)
        half = self.c * H
        pltpu.make_async_copy(self.b0.at[b, pl.ds(0, C)],
                              self.epre.at[pl.ds(e0, C), pl.ds(half, H)],
                              self.osem.at[b, 0]).wait()
        pltpu.make_async_copy(self.be.at[b], self.acc.at[self.dv2.at[b]],
                              self.osem.at[b, 1]).wait()


class _S2Pipe(_EdgePipe):
    def _d_descs(self, k, b):
        e0 = jnp.minimum(self._e0(k), E - C)
        half = self.c * H
        return [
            (self.ta.at[self.sa.at[b]], self.b0.at[b], self.dsem.at[b, 0]),
            (self.ex.at[pl.ds(e0, C), pl.ds(half, H)], self.be.at[b],
             self.dsem.at[b, 1]),
        ]

    def comp(self, b):
        _sub_idx(self.dv2.at[b], self.da.at[b], self.c * N)

        @plsc.parallel_loop(0, C, unroll=4)
        def _(r):
            for j in range(H // 16):
                sl = pl.ds(j * 16, 16)
                self.b0[b, r, sl] = self.b0[b, r, sl] * jnp.exp(self.be[b, r, sl])

    def o_go(self, k, b):
        pltpu.async_copy(self.b0.at[b], self.acc.at[self.dv2.at[b]],
                         self.osem.at[b, 0], add=True)

    def o_wait(self, k, b):
        pltpu.make_async_copy(self.b0.at[b], self.acc.at[self.dv2.at[b]],
                              self.osem.at[b, 0]).wait()


def _sc_edge_kernel(gidx_hbm, w01_hbm, ex_hbm, zero_hbm, epre_hbm, denom_hbm,
                    acc, si, dv2, bg, be, isem, dsem, osem):
    c = lax.axis_index("c")
    s = lax.axis_index("s")

    @pl.when(s == 0)
    def _():
        pltpu.sync_copy(zero_hbm, acc)
    plsc.subcore_barrier()

    p = _S1Pipe(c, s, gidx_hbm, None, None, w01_hbm, None, ex_hbm,
                si, None, None, dv2, bg, None, be, isem, dsem, osem, acc)
    p.epre = epre_hbm
    p.run()

    plsc.subcore_barrier()

    @pl.when(s == 0)
    def _():
        pltpu.sync_copy(acc, denom_hbm.at[c])


def _sc_edge(gidx, w01cat, ex, zeros):
    return pl.kernel(
        _sc_edge_kernel,
        out_type=[jax.ShapeDtypeStruct((EP, D), jnp.float32),
                  jax.ShapeDtypeStruct((2, NACC, H), jnp.float32)],
        mesh=_MESH,
        scratch_types=[
            pltpu.VMEM_SHARED((NACC, H), jnp.float32),
            pltpu.VMEM((2, 2 * C), jnp.int32),
            pltpu.VMEM((2, C), jnp.int32),
            pltpu.VMEM((2, 2 * C, H), jnp.float32),
            pltpu.VMEM((2, C, H), jnp.float32),
            pltpu.SemaphoreType.DMA((2, 1)),
            pltpu.SemaphoreType.DMA((2, 2)),
            pltpu.SemaphoreType.DMA((2, 2)),
        ],
    )(gidx, w01cat, ex, zeros)


def _sc_node_kernel(sadj_hbm, dadj_hbm, w3_hbm, ex_hbm, zero_hbm, ntmp_hbm,
                    acc, sa, da, dv2, b0, be, isem, dsem, osem):
    c = lax.axis_index("c")
    s = lax.axis_index("s")

    @pl.when(s == 0)
    def _():
        pltpu.sync_copy(zero_hbm, acc)
    plsc.subcore_barrier()

    p = _S2Pipe(c, s, sadj_hbm, dadj_hbm, None, w3_hbm, None, ex_hbm,
                sa, da, None, dv2, b0, None, be, isem, dsem, osem, acc)
    p.run()

    plsc.subcore_barrier()

    @pl.when(s == 0)
    def _():
        pltpu.sync_copy(acc, ntmp_hbm.at[c])


def _sc_node(sadj, dadj, w3cat, ex, zeros):
    return pl.kernel(
        _sc_node_kernel,
        out_type=jax.ShapeDtypeStruct((2, NACC, H), jnp.float32),
        mesh=_MESH,
        scratch_types=[
            pltpu.VMEM_SHARED((NACC, H), jnp.float32),
            pltpu.VMEM((2, C), jnp.int32),
            pltpu.VMEM((2, C), jnp.int32),
            pltpu.VMEM((2, C), jnp.int32),
            pltpu.VMEM((2, C, H), jnp.float32),
            pltpu.VMEM((2, C, H), jnp.float32),
            pltpu.SemaphoreType.DMA((2, 2)),
            pltpu.SemaphoreType.DMA((2, 2)),
            pltpu.SemaphoreType.DMA((2, 1)),
        ],
    )(sadj, dadj, w3cat, ex, zeros)


def _halves_cat(x):
    return jnp.concatenate([x[:, :H], x[:, H:]], axis=0)


def kernel(n_feat, e_feat, edge_index, W0, W1, W2, W3, W4,
           gamma_e, beta_e, gamma_n, beta_n):
    src = edge_index[0]
    dst = edge_index[1]
    zp = jnp.zeros((P,), jnp.int32)
    sadj = jnp.concatenate([src, zp, src + N, zp + N])
    dadj = jnp.concatenate([dst, zp + N, dst + N, zp + 2 * N])

    wnt = jnp.concatenate([W0, W1, W2, W3], axis=0).T
    hcat = _node_matmuls(n_feat, wnt)
    w0h, w1h, w2h, w3h = (hcat[:, :D], hcat[:, D:2 * D],
                          hcat[:, 2 * D:3 * D], hcat[:, 3 * D:])

    zeros = jnp.zeros((NACC, H), jnp.float32)
    w1cat = jnp.concatenate([_halves_cat(w1h), jnp.zeros((16, H))], axis=0)
    w01cat = jnp.concatenate([_halves_cat(w0h), w1cat], axis=0)
    gidx = jnp.concatenate([sadj.reshape(2, EP // C, C),
                            (dadj + 2 * N).reshape(2, EP // C, C)], axis=2)
    e_pre, denom = _sc_edge(gidx, w01cat, e_feat, zeros)

    w4e = _edge_mm(e_feat, W4.T)

    ntmp_h = _sc_node(sadj, dadj, _halves_cat(w3h), e_feat, zeros)

    stats, s_bf16 = _edge_stats(e_pre, w4e)
    new_e = _edge_update(stats, s_bf16, e_feat,
                         gamma_e.reshape(1, D), beta_e.reshape(1, D))

    nacc = ntmp_h[:, :N, :].transpose(1, 0, 2).reshape(N, D)
    dn = denom[:, :N, :].transpose(1, 0, 2).reshape(N, D)
    beta_n2 = (beta_n + new_e[0, :1] * 0.0).reshape(1, D)
    new_h = _node_update(nacc, dn, w2h, n_feat,
                         gamma_n.reshape(1, D), beta_n2)
    return (new_h, new_e)


# speedup vs baseline: 1.3108x; 1.0160x over previous
import functools

import jax
import jax.numpy as jnp
from jax import lax
from jax.experimental import pallas as pl
from jax.experimental.pallas import tpu as pltpu
from jax.experimental.pallas import tpu_sc as plsc

N = 10000
E = 160000
D = 256
H = 128
EPS = 1e-5

C = 64
TILES = 16
EDGES_PER_TILE = 10112
EP = EDGES_PER_TILE * TILES
P = EP - E
CHUNKS = EDGES_PER_TILE // C
NACC = N + 16


def _mm_kernel(x_ref, w_ref, o_ref):
    o_ref[...] = jnp.dot(x_ref[...], w_ref[...],
                         preferred_element_type=jnp.float32)


def _node_matmuls(n_feat, wnt):
    return pl.pallas_call(
        _mm_kernel,
        grid=(25,),
        in_specs=[pl.BlockSpec((400, D), lambda i: (i, 0)),
                  pl.BlockSpec((D, 4 * D), lambda i: (0, 0))],
        out_specs=pl.BlockSpec((400, 4 * D), lambda i: (i, 0)),
        out_shape=jax.ShapeDtypeStruct((N, 4 * D), jnp.float32),
    )(n_feat, wnt)


def _bf16_mm_kernel(x_ref, w_ref, o_ref):
    o_ref[...] = jnp.dot(x_ref[...].astype(jnp.bfloat16), w_ref[...],
                         preferred_element_type=jnp.float32
                         ).astype(jnp.bfloat16)


def _edge_mm(e_feat, w4t):
    return pl.pallas_call(
        _bf16_mm_kernel,
        grid=(160,),
        in_specs=[pl.BlockSpec((1000, D), lambda i: (i, 0)),
                  pl.BlockSpec((D, D), lambda i: (0, 0))],
        out_specs=pl.BlockSpec((1000, D), lambda i: (i, 0)),
        out_shape=jax.ShapeDtypeStruct((E, D), jnp.bfloat16),
    )(e_feat, w4t.astype(jnp.bfloat16))


def _estats_kernel(ep_ref, w4_ref, st_ref, s_ref):
    s = ep_ref[...] + w4_ref[...].astype(jnp.float32)
    ps = jnp.sum(s, axis=0)
    pq = jnp.sum(s * s, axis=0)
    z = jnp.zeros((6, D), jnp.float32)
    st_ref[...] = jnp.concatenate([ps[None], pq[None], z], axis=0)
    s_ref[...] = s.astype(jnp.bfloat16)


def _edge_stats(e_pre, w4e):
    return pl.pallas_call(
        _estats_kernel,
        grid=(160,),
        in_specs=[pl.BlockSpec((1000, D), lambda i: (i, 0)),
                  pl.BlockSpec((1000, D), lambda i: (i, 0))],
        out_specs=[pl.BlockSpec((8, D), lambda i: (i, 0)),
                   pl.BlockSpec((1000, D), lambda i: (i, 0))],
        out_shape=[jax.ShapeDtypeStruct((160 * 8, D), jnp.float32),
                   jax.ShapeDtypeStruct((E, D), jnp.bfloat16)],
    )(e_pre, w4e)


def _newe_kernel(st_ref, s_ref, ef_ref, g_ref, b_ref, o_ref):
    st = jnp.sum(st_ref[...].reshape(160, 8, D), axis=0)
    mean = st[0:1] / E
    var = st[1:2] / E - mean * mean
    inv = lax.rsqrt(var + EPS)
    s = s_ref[...].astype(jnp.float32)
    xn = (s - mean) * inv * g_ref[...] + b_ref[...]
    o_ref[...] = jnp.maximum(xn, 0.0) + ef_ref[...]


def _edge_update(stats, s_bf16, e_feat, gamma_e, beta_e):
    return pl.pallas_call(
        _newe_kernel,
        grid=(160,),
        in_specs=[pl.BlockSpec((160 * 8, D), lambda i: (0, 0)),
                  pl.BlockSpec((1000, D), lambda i: (i, 0)),
                  pl.BlockSpec((1000, D), lambda i: (i, 0)),
                  pl.BlockSpec((1, D), lambda i: (0, 0)),
                  pl.BlockSpec((1, D), lambda i: (0, 0))],
        out_specs=pl.BlockSpec((1000, D), lambda i: (i, 0)),
        out_shape=jax.ShapeDtypeStruct((E, D), jnp.float32),
    )(stats, s_bf16, e_feat, gamma_e, beta_e)


_HALF_SPECS = [
    pl.BlockSpec((1, 1000, H), lambda i: (0, i, 0)),
    pl.BlockSpec((1, 1000, H), lambda i: (1, i, 0)),
]


def _ntmp(na0, na1, dn0, dn1, w2):
    num = jnp.concatenate([na0[0], na1[0]], axis=1)
    den = jnp.concatenate([dn0[0], dn1[0]], axis=1)
    return num / jnp.maximum(den, 1e-30) + w2


def _nstats_kernel(na0, na1, dn0, dn1, w2_ref, st_ref):
    s = _ntmp(na0, na1, dn0, dn1, w2_ref[...])
    ps = jnp.sum(s, axis=0)
    pq = jnp.sum(s * s, axis=0)
    z = jnp.zeros((6, D), jnp.float32)
    st_ref[...] = jnp.concatenate([ps[None], pq[None], z], axis=0)


def _node_stats(ntmp_h, denom, w2h):
    return pl.pallas_call(
        _nstats_kernel,
        grid=(10,),
        in_specs=[*_HALF_SPECS, *_HALF_SPECS,
                  pl.BlockSpec((1000, D), lambda i: (i, 0))],
        out_specs=pl.BlockSpec((8, D), lambda i: (i, 0)),
        out_shape=jax.ShapeDtypeStruct((10 * 8, D), jnp.float32),
    )(ntmp_h, ntmp_h, denom, denom, w2h)


def _newh_kernel(st_ref, na0, na1, dn0, dn1, w2_ref, nf_ref, g_ref, b_ref,
                 o_ref):
    st = jnp.sum(st_ref[...].reshape(10, 8, D), axis=0)
    mean = st[0:1] / N
    var = st[1:2] / N - mean * mean
    inv = lax.rsqrt(var + EPS)
    s = _ntmp(na0, na1, dn0, dn1, w2_ref[...])
    xn = (s - mean) * inv * g_ref[...] + b_ref[...]
    o_ref[...] = jnp.maximum(xn, 0.0) + nf_ref[...]


def _node_update(ntmp_h, denom, w2h, n_feat, gamma_n, beta_n):
    stats = _node_stats(ntmp_h, denom, w2h)
    return pl.pallas_call(
        _newh_kernel,
        grid=(10,),
        in_specs=[pl.BlockSpec((10 * 8, D), lambda i: (0, 0)),
                  *_HALF_SPECS, *_HALF_SPECS,
                  pl.BlockSpec((1000, D), lambda i: (i, 0)),
                  pl.BlockSpec((1000, D), lambda i: (i, 0)),
                  pl.BlockSpec((1, D), lambda i: (0, 0)),
                  pl.BlockSpec((1, D), lambda i: (0, 0))],
        out_specs=pl.BlockSpec((1000, D), lambda i: (i, 0)),
        out_shape=jax.ShapeDtypeStruct((N, D), jnp.float32),
    )(stats, ntmp_h, ntmp_h, denom, denom, w2h, n_feat, gamma_n, beta_n)


_MESH = plsc.VectorSubcoreMesh(core_axis_name="c", subcore_axis_name="s")

PAIRS = CHUNKS // 2


def _sub_idx(dst_ref, src_ref, off):
    for v in range(C // 16):
        sl = pl.ds(v * 16, 16)
        dst_ref[sl] = src_ref[sl] - off


class _EdgePipe:
    def __init__(self, c, s, sadj, dadj, dstr, ta_hbm, tb_hbm, ex_hbm,
                 sa, da, dv, dv2, b0, b1, be, isem, dsem, osem, acc):
        self.c, self.s = c, s
        self.sadj, self.dadj, self.dstr = sadj, dadj, dstr
        self.ta, self.tb, self.ex = ta_hbm, tb_hbm, ex_hbm
        self.sa, self.da, self.dv, self.dv2 = sa, da, dv, dv2
        self.b0, self.b1, self.be = b0, b1, be
        self.isem, self.dsem, self.osem = isem, dsem, osem
        self.acc = acc

    def _e0(self, k):
        return self.s * EDGES_PER_TILE + k * C

    def _i_descs(self, k, b):
        e0 = self._e0(k)
        ge = self.c * EP + e0
        return [
            (self.sadj.at[pl.ds(ge, C)], self.sa.at[b], self.isem.at[b, 0]),
            (self.dadj.at[pl.ds(ge, C)], self.da.at[b], self.isem.at[b, 1]),
        ]

    def _d_descs(self, k, b):
        e0 = jnp.minimum(self._e0(k), E - C)
        half = self.c * H
        return [
            (self.ta.at[self.sa.at[b]], self.b0.at[b], self.dsem.at[b, 0]),
            (self.tb.at[self.da.at[b]], self.b1.at[b], self.dsem.at[b, 1]),
            (self.ex.at[pl.ds(e0, C), pl.ds(half, H)], self.be.at[b],
             self.dsem.at[b, 2]),
        ]

    def i_start(self, k, b):
        for sd in self._i_descs(k, b):
            pltpu.async_copy(*sd)

    def i_wait(self, k, b):
        for sd in self._i_descs(k, b):
            pltpu.make_async_copy(*sd).wait()

    def d_start(self, k, b):
        for sd in self._d_descs(k, b):
            pltpu.async_copy(*sd)

    def d_wait(self, k, b):
        for sd in self._d_descs(k, b):
            pltpu.make_async_copy(*sd).wait()

    def run(self):
        self.i_start(0, 0)

        def pair(g, _):
            k0 = 2 * g
            self.i_wait(k0, 0)

            @pl.when(g >= 1)
            def _():
                self.o_wait(k0 - 2, 0)
            self.d_start(k0, 0)

            @pl.when(g >= 1)
            def _():
                self.d_wait(k0 - 1, 1)
                self.comp(1)
                self.o_go(k0 - 1, 1)
            self.i_start(k0 + 1, 1)

            k1 = k0 + 1
            self.i_wait(k1, 1)

            @pl.when(g >= 1)
            def _():
                self.o_wait(k1 - 2, 1)
            self.d_start(k1, 1)

            self.d_wait(k0, 0)
            self.comp(0)
            self.o_go(k0, 0)

            @pl.when(k1 + 1 < CHUNKS)
            def _():
                self.i_start(k1 + 1, 0)
            return 0

        lax.fori_loop(0, PAIRS, pair, 0)
        self.d_wait(CHUNKS - 1, 1)
        self.comp(1)
        self.o_go(CHUNKS - 1, 1)
        self.o_wait(CHUNKS - 2, 0)
        self.o_wait(CHUNKS - 1, 1)


class _S1Pipe(_EdgePipe):
    def _i_descs(self, k, b):
        m = self.s * CHUNKS + k
        return [
            (self.sadj.at[self.c, m], self.sa.at[b], self.isem.at[b, 0]),
        ]

    def _d_descs(self, k, b):
        e0 = jnp.minimum(self._e0(k), E - C)
        half = self.c * H
        return [
            (self.ta.at[self.sa.at[b]], self.b0.at[b], self.dsem.at[b, 0]),
            (self.ex.at[pl.ds(e0, C), pl.ds(half, H)], self.be.at[b],
             self.dsem.at[b, 1]),
        ]

    def comp(self, b):
        _sub_idx(self.dv2.at[b], self.sa.at[b, pl.ds(C, C)],
                 2 * N + self.c * N)

        @plsc.parallel_loop(0, C, unroll=4)
        def _(r):
            for j in range(H // 16):
                sl = pl.ds(j * 16, 16)
                self.b0[b, r, sl] = self.b0[b, r, sl] + self.b0[b, C + r, sl]
                self.be[b, r, sl] = jnp.exp(self.be[b, r, sl])

    def o_go(self, k, b):
        e0 = self._e0(k)
        half = self.c * H
        pltpu.async_copy(self.b0.at[b, pl.ds(0, C)],
                         self.epre.at[pl.ds(e0, C), pl.ds(half, H)],
                         self.osem.at[b, 0])
        pltpu.async_copy(self.be.at[b], self.acc.at[self.dv2.at[b]],
                         self.osem.at[b, 1], add=True)

    def o_wait(self, k, b):
        e0 = self._e0(k)
        half = self.c * H
        pltpu.make_async_copy(self.b0.at[b, pl.ds(0, C)],
                              self.epre.at[pl.ds(e0, C), pl.ds(half, H)],
                              self.osem.at[b, 0]).wait()
        pltpu.make_async_copy(self.be.at[b], self.acc.at[self.dv2.at[b]],
                              self.osem.at[b, 1]).wait()


class _S2Pipe(_EdgePipe):
    def _d_descs(self, k, b):
        e0 = jnp.minimum(self._e0(k), E - C)
        half = self.c * H
        return [
            (self.ta.at[self.sa.at[b]], self.b0.at[b], self.dsem.at[b, 0]),
            (self.ex.at[pl.ds(e0, C), pl.ds(half, H)], self.be.at[b],
             self.dsem.at[b, 1]),
        ]

    def comp(self, b):
        _sub_idx(self.dv2.at[b], self.da.at[b], self.c * N)

        @plsc.parallel_loop(0, C, unroll=4)
        def _(r):
            for j in range(H // 16):
                sl = pl.ds(j * 16, 16)
                self.b0[b, r, sl] = self.b0[b, r, sl] * jnp.exp(self.be[b, r, sl])

    def o_go(self, k, b):
        pltpu.async_copy(self.b0.at[b], self.acc.at[self.dv2.at[b]],
                         self.osem.at[b, 0], add=True)

    def o_wait(self, k, b):
        pltpu.make_async_copy(self.b0.at[b], self.acc.at[self.dv2.at[b]],
                              self.osem.at[b, 0]).wait()


def _sc_edge_kernel(gidx_hbm, w01_hbm, ex_hbm, zero_hbm, epre_hbm, denom_hbm,
                    acc, si, dv2, bg, be, isem, dsem, osem):
    c = lax.axis_index("c")
    s = lax.axis_index("s")

    @pl.when(s == 0)
    def _():
        pltpu.sync_copy(zero_hbm, acc)
    plsc.subcore_barrier()

    p = _S1Pipe(c, s, gidx_hbm, None, None, w01_hbm, None, ex_hbm,
                si, None, None, dv2, bg, None, be, isem, dsem, osem, acc)
    p.epre = epre_hbm
    p.run()

    plsc.subcore_barrier()

    @pl.when(s == 0)
    def _():
        pltpu.sync_copy(acc, denom_hbm.at[c])


def _sc_edge(gidx, w01cat, ex, zeros):
    return pl.kernel(
        _sc_edge_kernel,
        out_type=[jax.ShapeDtypeStruct((EP, D), jnp.float32),
                  jax.ShapeDtypeStruct((2, NACC, H), jnp.float32)],
        mesh=_MESH,
        scratch_types=[
            pltpu.VMEM_SHARED((NACC, H), jnp.float32),
            pltpu.VMEM((2, 2 * C), jnp.int32),
            pltpu.VMEM((2, C), jnp.int32),
            pltpu.VMEM((2, 2 * C, H), jnp.float32),
            pltpu.VMEM((2, C, H), jnp.float32),
            pltpu.SemaphoreType.DMA((2, 1)),
            pltpu.SemaphoreType.DMA((2, 2)),
            pltpu.SemaphoreType.DMA((2, 2)),
        ],
    )(gidx, w01cat, ex, zeros)


def _sc_node_kernel(sadj_hbm, dadj_hbm, w3_hbm, ex_hbm, zero_hbm, ntmp_hbm,
                    acc, sa, da, dv2, b0, be, isem, dsem, osem):
    c = lax.axis_index("c")
    s = lax.axis_index("s")

    @pl.when(s == 0)
    def _():
        pltpu.sync_copy(zero_hbm, acc)
    plsc.subcore_barrier()

    p = _S2Pipe(c, s, sadj_hbm, dadj_hbm, None, w3_hbm, None, ex_hbm,
                sa, da, None, dv2, b0, None, be, isem, dsem, osem, acc)
    p.run()

    plsc.subcore_barrier()

    @pl.when(s == 0)
    def _():
        pltpu.sync_copy(acc, ntmp_hbm.at[c])


def _sc_node(sadj, dadj, w3cat, ex, zeros):
    return pl.kernel(
        _sc_node_kernel,
        out_type=jax.ShapeDtypeStruct((2, NACC, H), jnp.float32),
        mesh=_MESH,
        scratch_types=[
            pltpu.VMEM_SHARED((NACC, H), jnp.float32),
            pltpu.VMEM((2, C), jnp.int32),
            pltpu.VMEM((2, C), jnp.int32),
            pltpu.VMEM((2, C), jnp.int32),
            pltpu.VMEM((2, C, H), jnp.float32),
            pltpu.VMEM((2, C, H), jnp.float32),
            pltpu.SemaphoreType.DMA((2, 2)),
            pltpu.SemaphoreType.DMA((2, 2)),
            pltpu.SemaphoreType.DMA((2, 1)),
        ],
    )(sadj, dadj, w3cat, ex, zeros)


def _halves_cat(x):
    return jnp.concatenate([x[:, :H], x[:, H:]], axis=0)


def kernel(n_feat, e_feat, edge_index, W0, W1, W2, W3, W4,
           gamma_e, beta_e, gamma_n, beta_n):
    src = edge_index[0]
    dst = edge_index[1]
    zp = jnp.zeros((P,), jnp.int32)
    sadj = jnp.concatenate([src, zp, src + N, zp + N])
    dadj = jnp.concatenate([dst, zp + N, dst + N, zp + 2 * N])

    wnt = jnp.concatenate([W0, W1, W2, W3], axis=0).T
    hcat = _node_matmuls(n_feat, wnt)
    w0h, w1h, w2h, w3h = (hcat[:, :D], hcat[:, D:2 * D],
                          hcat[:, 2 * D:3 * D], hcat[:, 3 * D:])

    zeros = jnp.zeros((NACC, H), jnp.float32)
    w1cat = jnp.concatenate([_halves_cat(w1h), jnp.zeros((16, H))], axis=0)
    w01cat = jnp.concatenate([_halves_cat(w0h), w1cat], axis=0)
    gidx = jnp.concatenate([sadj.reshape(2, EP // C, C),
                            (dadj + 2 * N).reshape(2, EP // C, C)], axis=2)
    e_pre, denom = _sc_edge(gidx, w01cat, e_feat, zeros)

    w4e = _edge_mm(e_feat, W4.T)

    ntmp_h = _sc_node(sadj, dadj, _halves_cat(w3h), e_feat, zeros)

    stats, s_bf16 = _edge_stats(e_pre, w4e)
    new_e = _edge_update(stats, s_bf16, e_feat,
                         gamma_e.reshape(1, D), beta_e.reshape(1, D))

    beta_n2 = (beta_n + new_e[0, :1] * 0.0).reshape(1, D)
    new_h = _node_update(ntmp_h, denom, w2h, n_feat,
                         gamma_n.reshape(1, D), beta_n2)
    return (new_h, new_e)
